# Initial kernel scaffold; baseline (speedup 1.0000x reference)
import jax
import jax.numpy as jnp
from jax.experimental import pallas as pl


def kernel(features_norm, edge_index, neg_sample_list, W1, b1, a1, W2, b2, a2, a3, Wd1, bd1, Wd2, bd2):
    raise NotImplementedError("write your pallas kernel here")


# speedup vs baseline: 2.9281x; 2.9281x over previous
import functools

import jax
import jax.numpy as jnp
from jax.experimental import pallas as pl

N = 10000
E = 160000
F_IN = 128
F_H = 64
N_NEG = 5

BLK = 512


def _prelu(x, a):
    return jnp.where(x >= 0, x, a * x)


def _adj_body(a_ref, b_ref, o_ref):
    a = a_ref[...]
    b = b_ref[...]
    acc = jax.lax.dot_general(a, b, (((1,), (1,)), ((), ())),
                              preferred_element_type=jnp.float32)
    o_ref[...] = jax.nn.sigmoid(acc)


def _adj_rebuilt(h2):
    nb = pl.cdiv(N, BLK)
    return pl.pallas_call(
        _adj_body,
        grid=(nb, nb),
        in_specs=[
            pl.BlockSpec((BLK, F_H), lambda i, j: (i, 0)),
            pl.BlockSpec((BLK, F_H), lambda i, j: (j, 0)),
        ],
        out_specs=pl.BlockSpec((BLK, BLK), lambda i, j: (i, j)),
        out_shape=jax.ShapeDtypeStruct((N, N), jnp.float32),
    )(h2, h2)


def kernel(features_norm, edge_index, neg_sample_list, W1, b1, a1, W2, b2, a2, a3, Wd1, bd1, Wd2, bd2):
    x = jnp.squeeze(features_norm, 0)
    src, dst = edge_index[0], edge_index[1]

    cnt = jnp.zeros((N,), jnp.float32).at[dst].add(1.0)
    dinv = jax.lax.rsqrt(cnt + 1.0)
    oinv = 1.0 / jnp.maximum(cnt, 1.0)

    hw1 = x @ W1 + b1
    t1 = hw1 * dinv[:, None]
    s_norm1 = jnp.zeros((N, F_H), jnp.float32).at[dst].add(t1[src])
    h1 = _prelu(dinv[:, None] * (s_norm1 + t1), a1)

    hw2 = h1 @ W2 + b2
    t2 = hw2 * dinv[:, None]
    s_norm2 = jnp.zeros((N, F_H), jnp.float32).at[dst].add(t2[src])
    h2 = _prelu(dinv[:, None] * (s_norm2 + t2), a2)

    s_orig = jnp.zeros((N, F_H), jnp.float32).at[dst].add(hw1[src])
    h_neighbor = _prelu(oinv[:, None] * s_orig, a3)

    y1 = x @ Wd1
    y2 = h_neighbor @ Wd2
    mi_pos = jax.nn.sigmoid(jnp.sum(y1 * h2, axis=-1) + bd1)
    mi_neg = jax.nn.sigmoid(jnp.sum(y1[neg_sample_list] * h2[None], axis=-1) + bd1)
    local_mi_pos = jax.nn.sigmoid(jnp.sum(y2 * h2, axis=-1) + bd2)
    local_mi_neg = jax.nn.sigmoid(jnp.sum(y2[neg_sample_list] * h2[None], axis=-1) + bd2)

    adj_rebuilt = _adj_rebuilt(h2)
    return (mi_pos, mi_neg, local_mi_pos, local_mi_neg, adj_rebuilt)


# speedup vs baseline: 5.9324x; 2.0260x over previous
import functools

import jax
import jax.numpy as jnp
from jax import lax
from jax.experimental import pallas as pl
from jax.experimental.pallas import tpu as pltpu
from jax.experimental.pallas import tpu_sc as plsc

N = 10000
E = 160000
F_IN = 128
F_H = 64
N_NEG = 5

NC, NS = 2, 16
NW = NC * NS
NPAD = 10240
TRASH = N + 100
CH = 128
EPAD = 163840
CPW = EPAD // (NW * CH)
ROWS_PT = NPAD // NS

BLK = 512


def _prelu(x, a):
    return jnp.where(x >= 0, x, a * x)


def _sc_scatter_rows(D):
    mesh = plsc.VectorSubcoreMesh(core_axis_name="c", subcore_axis_name="s")

    @functools.partial(
        pl.kernel,
        out_type=jax.ShapeDtypeStruct((NC, NPAD, D), jnp.float32),
        mesh=mesh,
        scratch_types=[
            pltpu.VMEM((CPW, CH), jnp.int32),
            pltpu.VMEM((CPW, CH), jnp.int32),
            pltpu.VMEM((CH, D), jnp.float32),
            pltpu.VMEM_SHARED((NPAD, D), jnp.float32),
            pltpu.SemaphoreType.DMA,
        ],
        compiler_params=pltpu.CompilerParams(use_tc_tiling_on_sc=False),
    )
    def k(table_hbm, src_hbm, dst_hbm, zeros_hbm, out_hbm, srcv, dstv, rows, acc, sem):
        cid = lax.axis_index("c")
        sid = lax.axis_index("s")
        wid = sid * NC + cid
        r0 = sid * ROWS_PT
        pltpu.sync_copy(zeros_hbm.at[pl.ds(r0, ROWS_PT)], acc.at[pl.ds(r0, ROWS_PT)])
        pltpu.sync_copy(src_hbm.at[pl.ds(wid * CPW, CPW)], srcv)
        pltpu.sync_copy(dst_hbm.at[pl.ds(wid * CPW, CPW)], dstv)
        plsc.subcore_barrier()

        def body(i, _):
            pltpu.async_copy(table_hbm.at[srcv.at[i]], rows, sem).wait()
            pltpu.sync_copy(rows, acc.at[dstv.at[i]], add=True)
            return 0

        lax.fori_loop(0, CPW, body, 0)
        plsc.subcore_barrier()
        pltpu.sync_copy(acc.at[pl.ds(r0, ROWS_PT)], out_hbm.at[cid, pl.ds(r0, ROWS_PT)])

    return k


def _sc_degree():
    mesh = plsc.VectorSubcoreMesh(core_axis_name="c", subcore_axis_name="s")

    @functools.partial(
        pl.kernel,
        out_type=jax.ShapeDtypeStruct((NC, NPAD, 16), jnp.float32),
        mesh=mesh,
        scratch_types=[
            pltpu.VMEM((CPW, CH), jnp.int32),
            pltpu.VMEM((CH, 16), jnp.float32),
            pltpu.VMEM_SHARED((NPAD, 16), jnp.float32),
        ],
        compiler_params=pltpu.CompilerParams(use_tc_tiling_on_sc=False),
    )
    def k(ones_hbm, dst_hbm, zeros_hbm, out_hbm, dstv, ones_v, acc):
        cid = lax.axis_index("c")
        sid = lax.axis_index("s")
        wid = sid * NC + cid
        r0 = sid * ROWS_PT
        pltpu.sync_copy(zeros_hbm.at[pl.ds(r0, ROWS_PT)], acc.at[pl.ds(r0, ROWS_PT)])
        pltpu.sync_copy(ones_hbm, ones_v)
        pltpu.sync_copy(dst_hbm.at[pl.ds(wid * CPW, CPW)], dstv)
        plsc.subcore_barrier()

        def body(i, _):
            pltpu.sync_copy(ones_v, acc.at[dstv.at[i]], add=True)
            return 0

        lax.fori_loop(0, CPW, body, 0)
        plsc.subcore_barrier()
        pltpu.sync_copy(acc.at[pl.ds(r0, ROWS_PT)], out_hbm.at[cid, pl.ds(r0, ROWS_PT)])

    return k


GCH = 80
GTOT = N_NEG * NPAD
GCPW = GTOT // (NW * GCH)


def _sc_gather_rows(D):
    mesh = plsc.VectorSubcoreMesh(core_axis_name="c", subcore_axis_name="s")

    @functools.partial(
        pl.kernel,
        out_type=jax.ShapeDtypeStruct((GTOT, D), jnp.float32),
        mesh=mesh,
        scratch_types=[
            pltpu.VMEM((GCPW, GCH), jnp.int32),
            pltpu.VMEM((GCH, D), jnp.float32),
            pltpu.SemaphoreType.DMA,
        ],
    )
    def k(table_hbm, idx_hbm, out_hbm, idxv, rows, sem):
        cid = lax.axis_index("c")
        sid = lax.axis_index("s")
        wid = sid * NC + cid
        pltpu.sync_copy(idx_hbm.at[wid], idxv)

        def body(i, _):
            pltpu.async_copy(table_hbm.at[idxv.at[i]], rows, sem).wait()
            pltpu.sync_copy(rows, out_hbm.at[pl.ds((wid * GCPW + i) * GCH, GCH)])
            return 0

        lax.fori_loop(0, GCPW, body, 0)

    return k


def _adj_body(a_ref, b_ref, o_ref):
    acc = lax.dot_general(a_ref[...], b_ref[...], (((1,), (1,)), ((), ())),
                          preferred_element_type=jnp.float32)
    o_ref[...] = jax.nn.sigmoid(acc)


def _adj_rebuilt(h2):
    nb = pl.cdiv(N, BLK)
    return pl.pallas_call(
        _adj_body,
        grid=(nb, nb),
        in_specs=[
            pl.BlockSpec((BLK, F_H), lambda i, j: (i, 0)),
            pl.BlockSpec((BLK, F_H), lambda i, j: (j, 0)),
        ],
        out_specs=pl.BlockSpec((BLK, BLK), lambda i, j: (i, j)),
        out_shape=jax.ShapeDtypeStruct((N, N), jnp.float32),
    )(h2, h2)


def kernel(features_norm, edge_index, neg_sample_list, W1, b1, a1, W2, b2, a2, a3, Wd1, bd1, Wd2, bd2):
    x = jnp.squeeze(features_norm, 0)
    src, dst = edge_index[0], edge_index[1]

    pad = EPAD - E
    srcp = jnp.concatenate([src, jnp.zeros((pad,), jnp.int32)]).reshape(EPAD // CH, CH)
    dstp = jnp.concatenate([dst, jnp.full((pad,), TRASH, jnp.int32)]).reshape(EPAD // CH, CH)

    z16 = jnp.zeros((NPAD, 16), jnp.float32)
    z64 = jnp.zeros((NPAD, F_H), jnp.float32)
    z128 = jnp.zeros((NPAD, 2 * F_H), jnp.float32)
    ones16 = jnp.ones((CH, 16), jnp.float32)

    cnt2 = _sc_degree()(ones16, dstp, z16)
    cnt = cnt2[0, :N, 0] + cnt2[1, :N, 0]
    dinv = lax.rsqrt(cnt + 1.0)
    oinv = 1.0 / jnp.maximum(cnt, 1.0)

    hw1 = x @ W1 + b1
    t1 = hw1 * dinv[:, None]
    tab1 = jnp.zeros((NPAD, 2 * F_H), jnp.float32).at[:N, :F_H].set(hw1).at[:N, F_H:].set(t1)

    s1 = _sc_scatter_rows(2 * F_H)(tab1, srcp, dstp, z128)
    s1sum = s1[0, :N] + s1[1, :N]
    s_orig = s1sum[:, :F_H]
    s_norm1 = s1sum[:, F_H:]
    h1 = _prelu(dinv[:, None] * (s_norm1 + t1), a1)
    h_neighbor = _prelu(oinv[:, None] * s_orig, a3)

    hw2 = h1 @ W2 + b2
    t2 = hw2 * dinv[:, None]
    tab2 = jnp.zeros((NPAD, F_H), jnp.float32).at[:N].set(t2)

    s2 = _sc_scatter_rows(F_H)(tab2, srcp, dstp, z64)
    s_norm2 = s2[0, :N] + s2[1, :N]
    h2 = _prelu(dinv[:, None] * (s_norm2 + t2), a2)

    y1 = x @ Wd1
    y2 = h_neighbor @ Wd2
    mi_pos = jax.nn.sigmoid(jnp.sum(y1 * h2, axis=-1) + bd1)
    local_mi_pos = jax.nn.sigmoid(jnp.sum(y2 * h2, axis=-1) + bd2)

    y12 = jnp.concatenate([y1, y2], axis=1)
    negp = jnp.pad(neg_sample_list, ((0, 0), (0, NPAD - N))).reshape(NW, GCPW, GCH)
    g = _sc_gather_rows(2 * F_H)(y12, negp)
    g = g.reshape(N_NEG, NPAD, 2 * F_H)[:, :N]
    mi_neg = jax.nn.sigmoid(jnp.sum(g[:, :, :F_H] * h2[None], axis=-1) + bd1)
    local_mi_neg = jax.nn.sigmoid(jnp.sum(g[:, :, F_H:] * h2[None], axis=-1) + bd2)

    adj_rebuilt = _adj_rebuilt(h2)
    return (mi_pos, mi_neg, local_mi_pos, local_mi_neg, adj_rebuilt)


# speedup vs baseline: 6.0768x; 1.0243x over previous
import functools

import jax
import jax.numpy as jnp
from jax import lax
from jax.experimental import pallas as pl
from jax.experimental.pallas import tpu as pltpu
from jax.experimental.pallas import tpu_sc as plsc

N = 10000
E = 160000
F_IN = 128
F_H = 64
N_NEG = 5

NC, NS = 2, 16
NW = NC * NS
NPAD = 10240
TRASH = N + 100
CH = 128
EPAD = 163840
CPW = EPAD // (NW * CH)
ROWS_PT = NPAD // NS

BLK = 512


def _prelu(x, a):
    return jnp.where(x >= 0, x, a * x)


def _sc_scatter_rows(D, ch, nb):
    mesh = plsc.VectorSubcoreMesh(core_axis_name="c", subcore_axis_name="s")
    cpw = EPAD // (NW * ch)
    n_groups = cpw // nb

    @functools.partial(
        pl.kernel,
        out_type=jax.ShapeDtypeStruct((NC, NPAD, D), jnp.float32),
        mesh=mesh,
        scratch_types=[
            pltpu.VMEM((cpw, ch), jnp.int32),
            pltpu.VMEM((cpw, ch), jnp.int32),
            pltpu.VMEM((nb, ch, D), jnp.float32),
            pltpu.VMEM_SHARED((NPAD, D), jnp.float32),
            pltpu.SemaphoreType.DMA,
            pltpu.SemaphoreType.DMA,
        ],
        compiler_params=pltpu.CompilerParams(use_tc_tiling_on_sc=False),
    )
    def k(table_hbm, src_hbm, dst_hbm, zeros_hbm, out_hbm, srcv, dstv, rows, acc, gsem, ssem):
        cid = lax.axis_index("c")
        sid = lax.axis_index("s")
        wid = sid * NC + cid
        r0 = sid * ROWS_PT
        pltpu.sync_copy(zeros_hbm.at[pl.ds(r0, ROWS_PT)], acc.at[pl.ds(r0, ROWS_PT)])
        pltpu.sync_copy(src_hbm.at[pl.ds(wid * cpw, cpw)], srcv)
        pltpu.sync_copy(dst_hbm.at[pl.ds(wid * cpw, cpw)], dstv)
        plsc.subcore_barrier()

        def group(g, _):
            c0 = g * nb
            gds = [pltpu.async_copy(table_hbm.at[srcv.at[c0 + b]], rows.at[b], gsem)
                   for b in range(nb)]
            for gd in gds:
                gd.wait()
            sds = [pltpu.async_copy(rows.at[b], acc.at[dstv.at[c0 + b]], ssem, add=True)
                   for b in range(nb)]
            for sd in sds:
                sd.wait()
            return 0

        lax.fori_loop(0, n_groups, group, 0)
        plsc.subcore_barrier()
        pltpu.sync_copy(acc.at[pl.ds(r0, ROWS_PT)], out_hbm.at[cid, pl.ds(r0, ROWS_PT)])

    return k


def _sc_degree():
    mesh = plsc.VectorSubcoreMesh(core_axis_name="c", subcore_axis_name="s")

    @functools.partial(
        pl.kernel,
        out_type=jax.ShapeDtypeStruct((NC, NPAD, 16), jnp.float32),
        mesh=mesh,
        scratch_types=[
            pltpu.VMEM((CPW, CH), jnp.int32),
            pltpu.VMEM((CH, 16), jnp.float32),
            pltpu.VMEM_SHARED((NPAD, 16), jnp.float32),
            pltpu.SemaphoreType.DMA,
        ],
        compiler_params=pltpu.CompilerParams(use_tc_tiling_on_sc=False),
    )
    def k(ones_hbm, dst_hbm, zeros_hbm, out_hbm, dstv, ones_v, acc, ssem):
        cid = lax.axis_index("c")
        sid = lax.axis_index("s")
        wid = sid * NC + cid
        r0 = sid * ROWS_PT
        pltpu.sync_copy(zeros_hbm.at[pl.ds(r0, ROWS_PT)], acc.at[pl.ds(r0, ROWS_PT)])
        pltpu.sync_copy(ones_hbm, ones_v)
        pltpu.sync_copy(dst_hbm.at[pl.ds(wid * CPW, CPW)], dstv)
        plsc.subcore_barrier()

        def group(g, _):
            sds = [pltpu.async_copy(ones_v, acc.at[dstv.at[g * 8 + b]], ssem, add=True)
                   for b in range(8)]
            for sd in sds:
                sd.wait()
            return 0

        lax.fori_loop(0, CPW // 8, group, 0)
        plsc.subcore_barrier()
        pltpu.sync_copy(acc.at[pl.ds(r0, ROWS_PT)], out_hbm.at[cid, pl.ds(r0, ROWS_PT)])

    return k


GCH = 80
GTOT = N_NEG * NPAD
GCPW = GTOT // (NW * GCH)
GNB = 5


def _sc_gather_rows(D):
    mesh = plsc.VectorSubcoreMesh(core_axis_name="c", subcore_axis_name="s")

    @functools.partial(
        pl.kernel,
        out_type=jax.ShapeDtypeStruct((GTOT, D), jnp.float32),
        mesh=mesh,
        scratch_types=[
            pltpu.VMEM((GCPW, GCH), jnp.int32),
            pltpu.VMEM((GNB, GCH, D), jnp.float32),
            pltpu.SemaphoreType.DMA,
            pltpu.SemaphoreType.DMA,
        ],
    )
    def k(table_hbm, idx_hbm, out_hbm, idxv, rows, gsem, ssem):
        cid = lax.axis_index("c")
        sid = lax.axis_index("s")
        wid = sid * NC + cid
        pltpu.sync_copy(idx_hbm.at[wid], idxv)

        def group(g, _):
            c0 = g * GNB
            gds = [pltpu.async_copy(table_hbm.at[idxv.at[c0 + b]], rows.at[b], gsem)
                   for b in range(GNB)]
            for gd in gds:
                gd.wait()
            sds = [pltpu.async_copy(rows.at[b],
                                    out_hbm.at[pl.ds((wid * GCPW + c0 + b) * GCH, GCH)],
                                    ssem)
                   for b in range(GNB)]
            for sd in sds:
                sd.wait()
            return 0

        lax.fori_loop(0, GCPW // GNB, group, 0)

    return k


def _adj_body(a_ref, b_ref, o_ref):
    acc = lax.dot_general(a_ref[...], b_ref[...], (((1,), (1,)), ((), ())),
                          preferred_element_type=jnp.float32)
    o_ref[...] = jax.nn.sigmoid(acc)


def _adj_rebuilt(h2):
    nb = pl.cdiv(N, BLK)
    return pl.pallas_call(
        _adj_body,
        grid=(nb, nb),
        in_specs=[
            pl.BlockSpec((BLK, F_H), lambda i, j: (i, 0)),
            pl.BlockSpec((BLK, F_H), lambda i, j: (j, 0)),
        ],
        out_specs=pl.BlockSpec((BLK, BLK), lambda i, j: (i, j)),
        out_shape=jax.ShapeDtypeStruct((N, N), jnp.float32),
    )(h2, h2)


def kernel(features_norm, edge_index, neg_sample_list, W1, b1, a1, W2, b2, a2, a3, Wd1, bd1, Wd2, bd2):
    x = jnp.squeeze(features_norm, 0)
    src, dst = edge_index[0], edge_index[1]

    pad = EPAD - E
    srcp = jnp.concatenate([src, jnp.zeros((pad,), jnp.int32)]).reshape(EPAD // CH, CH)
    dstp = jnp.concatenate([dst, jnp.full((pad,), TRASH, jnp.int32)]).reshape(EPAD // CH, CH)

    z16 = jnp.zeros((NPAD, 16), jnp.float32)
    z64 = jnp.zeros((NPAD, F_H), jnp.float32)
    z128 = jnp.zeros((NPAD, 2 * F_H), jnp.float32)
    ones16 = jnp.ones((CH, 16), jnp.float32)

    cnt2 = _sc_degree()(ones16, dstp, z16)
    cnt = cnt2[0, :N, 0] + cnt2[1, :N, 0]
    dinv = lax.rsqrt(cnt + 1.0)
    oinv = 1.0 / jnp.maximum(cnt, 1.0)

    hw1 = x @ W1 + b1
    t1 = hw1 * dinv[:, None]
    tab1 = jnp.zeros((NPAD, 2 * F_H), jnp.float32).at[:N, :F_H].set(hw1).at[:N, F_H:].set(t1)

    src64 = srcp.reshape(EPAD // 64, 64)
    dst64 = dstp.reshape(EPAD // 64, 64)
    s1 = _sc_scatter_rows(2 * F_H, 64, 4)(tab1, src64, dst64, z128)
    s1sum = s1[0, :N] + s1[1, :N]
    s_orig = s1sum[:, :F_H]
    s_norm1 = s1sum[:, F_H:]
    h1 = _prelu(dinv[:, None] * (s_norm1 + t1), a1)
    h_neighbor = _prelu(oinv[:, None] * s_orig, a3)

    hw2 = h1 @ W2 + b2
    t2 = hw2 * dinv[:, None]
    tab2 = jnp.zeros((NPAD, F_H), jnp.float32).at[:N].set(t2)

    s2 = _sc_scatter_rows(F_H, 64, 8)(tab2, src64, dst64, z64)
    s_norm2 = s2[0, :N] + s2[1, :N]
    h2 = _prelu(dinv[:, None] * (s_norm2 + t2), a2)

    y1 = x @ Wd1
    y2 = h_neighbor @ Wd2
    mi_pos = jax.nn.sigmoid(jnp.sum(y1 * h2, axis=-1) + bd1)
    local_mi_pos = jax.nn.sigmoid(jnp.sum(y2 * h2, axis=-1) + bd2)

    y12 = jnp.concatenate([y1, y2], axis=1)
    negp = jnp.pad(neg_sample_list, ((0, 0), (0, NPAD - N))).reshape(NW, GCPW, GCH)
    g = _sc_gather_rows(2 * F_H)(y12, negp)
    g = g.reshape(N_NEG, NPAD, 2 * F_H)[:, :N]
    mi_neg = jax.nn.sigmoid(jnp.sum(g[:, :, :F_H] * h2[None], axis=-1) + bd1)
    local_mi_neg = jax.nn.sigmoid(jnp.sum(g[:, :, F_H:] * h2[None], axis=-1) + bd2)

    adj_rebuilt = _adj_rebuilt(h2)
    return (mi_pos, mi_neg, local_mi_pos, local_mi_neg, adj_rebuilt)


# speedup vs baseline: 7.5616x; 1.2443x over previous
import functools

import jax
import jax.numpy as jnp
from jax import lax
from jax.experimental import pallas as pl
from jax.experimental.pallas import tpu as pltpu
from jax.experimental.pallas import tpu_sc as plsc

N = 10000
E = 160000
F_IN = 128
F_H = 64
N_NEG = 5

NC, NS = 2, 16
NW = NC * NS
NPAD = 10240
TRASH = N + 100
CH = 128
EPAD = 163840
CPW = EPAD // (NW * CH)
ROWS_PT = NPAD // NS

CHT = EPAD // 64
CPS = CHT // NS
HALF = CPS // 2

RB = 512
NRB = NPAD // RB
BM, BN = 512, 2048


def _prelu(x, a):
    return jnp.where(x >= 0, x, a * x)


def _sc_spmm_split(D, nb):
    mesh = plsc.VectorSubcoreMesh(core_axis_name="c", subcore_axis_name="s")

    @functools.partial(
        pl.kernel,
        out_type=jax.ShapeDtypeStruct((NC, NPAD, D), jnp.float32),
        mesh=mesh,
        scratch_types=[
            pltpu.VMEM((HALF, 64), jnp.int32),
            pltpu.VMEM((HALF, 64), jnp.int32),
            pltpu.VMEM((nb, 64, D), jnp.float32),
            pltpu.VMEM_SHARED((NPAD, D), jnp.float32),
            pltpu.SemaphoreType.DMA,
            pltpu.SemaphoreType.DMA,
        ],
        compiler_params=pltpu.CompilerParams(use_tc_tiling_on_sc=False),
    )
    def k(table_hbm, srcoff_hbm, dst_hbm, zeros_hbm, out_hbm, srcv, dstv, rows, acc, gsem, ssem):
        cid = lax.axis_index("c")
        sid = lax.axis_index("s")
        r0 = sid * ROWS_PT
        pltpu.sync_copy(zeros_hbm.at[pl.ds(r0, ROWS_PT)], acc.at[pl.ds(r0, ROWS_PT)])
        plsc.subcore_barrier()

        c_base = sid * CPS
        for h in range(2):
            pltpu.sync_copy(srcoff_hbm.at[cid, pl.ds(c_base + h * HALF, HALF)], srcv)
            pltpu.sync_copy(dst_hbm.at[pl.ds(c_base + h * HALF, HALF)], dstv)

            def group(g, _):
                c0 = g * nb
                gds = [pltpu.async_copy(table_hbm.at[srcv.at[c0 + b]], rows.at[b], gsem)
                       for b in range(nb)]
                for gd in gds:
                    gd.wait()
                sds = [pltpu.async_copy(rows.at[b], acc.at[dstv.at[c0 + b]], ssem, add=True)
                       for b in range(nb)]
                for sd in sds:
                    sd.wait()
                return 0

            lax.fori_loop(0, HALF // nb, group, 0)
        plsc.subcore_barrier()
        pltpu.sync_copy(acc.at[pl.ds(r0, ROWS_PT)], out_hbm.at[cid, pl.ds(r0, ROWS_PT)])

    return k


def _sc_degree():
    mesh = plsc.VectorSubcoreMesh(core_axis_name="c", subcore_axis_name="s")

    @functools.partial(
        pl.kernel,
        out_type=jax.ShapeDtypeStruct((NC, NPAD, 16), jnp.float32),
        mesh=mesh,
        scratch_types=[
            pltpu.VMEM((CPW, CH), jnp.int32),
            pltpu.VMEM((CH, 16), jnp.float32),
            pltpu.VMEM_SHARED((NPAD, 16), jnp.float32),
            pltpu.SemaphoreType.DMA,
        ],
        compiler_params=pltpu.CompilerParams(use_tc_tiling_on_sc=False),
    )
    def k(ones_hbm, dst_hbm, zeros_hbm, out_hbm, dstv, ones_v, acc, ssem):
        cid = lax.axis_index("c")
        sid = lax.axis_index("s")
        wid = sid * NC + cid
        r0 = sid * ROWS_PT
        pltpu.sync_copy(zeros_hbm.at[pl.ds(r0, ROWS_PT)], acc.at[pl.ds(r0, ROWS_PT)])
        pltpu.sync_copy(ones_hbm, ones_v)
        pltpu.sync_copy(dst_hbm.at[pl.ds(wid * CPW, CPW)], dstv)
        plsc.subcore_barrier()

        def group(g, _):
            sds = [pltpu.async_copy(ones_v, acc.at[dstv.at[g * 8 + b]], ssem, add=True)
                   for b in range(8)]
            for sd in sds:
                sd.wait()
            return 0

        lax.fori_loop(0, CPW // 8, group, 0)
        plsc.subcore_barrier()
        pltpu.sync_copy(acc.at[pl.ds(r0, ROWS_PT)], out_hbm.at[cid, pl.ds(r0, ROWS_PT)])

    return k


GCH = 80
GTOT = N_NEG * NPAD
GCPW = GTOT // (NW * GCH)
GNB = 5


def _sc_gather_rows(D):
    mesh = plsc.VectorSubcoreMesh(core_axis_name="c", subcore_axis_name="s")

    @functools.partial(
        pl.kernel,
        out_type=jax.ShapeDtypeStruct((GTOT, D), jnp.float32),
        mesh=mesh,
        scratch_types=[
            pltpu.VMEM((GCPW, GCH), jnp.int32),
            pltpu.VMEM((GNB, GCH, D), jnp.float32),
            pltpu.SemaphoreType.DMA,
            pltpu.SemaphoreType.DMA,
        ],
    )
    def k(table_hbm, idx_hbm, out_hbm, idxv, rows, gsem, ssem):
        cid = lax.axis_index("c")
        sid = lax.axis_index("s")
        wid = sid * NC + cid
        pltpu.sync_copy(idx_hbm.at[wid], idxv)

        def group(g, _):
            c0 = g * GNB
            gds = [pltpu.async_copy(table_hbm.at[idxv.at[c0 + b]], rows.at[b], gsem)
                   for b in range(GNB)]
            for gd in gds:
                gd.wait()
            sds = [pltpu.async_copy(rows.at[b],
                                    out_hbm.at[pl.ds((wid * GCPW + c0 + b) * GCH, GCH)],
                                    ssem)
                   for b in range(GNB)]
            for sd in sds:
                sd.wait()
            return 0

        lax.fori_loop(0, GCPW // GNB, group, 0)

    return k


def _mm(a, b):
    return lax.dot_general(a, b, (((1,), (0,)), ((), ())),
                           preferred_element_type=jnp.float32)


def _layer1_body(x_ref, w1_ref, b1_ref, wd1_ref, d_ref, tab_ref, y1_ref):
    xb = x_ref[...]
    hw1 = _mm(xb, w1_ref[...]) + b1_ref[...]
    d = d_ref[...]
    tab_ref[0] = hw1
    tab_ref[1] = hw1 * d
    y1_ref[...] = _mm(xb, wd1_ref[...])


def _tc_layer1(xp, W1, b1, Wd1, dinv1):
    return pl.pallas_call(
        _layer1_body,
        grid=(NRB,),
        in_specs=[
            pl.BlockSpec((RB, F_IN), lambda i: (i, 0)),
            pl.BlockSpec((F_IN, F_H), lambda i: (0, 0)),
            pl.BlockSpec((1, F_H), lambda i: (0, 0)),
            pl.BlockSpec((F_IN, F_H), lambda i: (0, 0)),
            pl.BlockSpec((RB, 1), lambda i: (i, 0)),
        ],
        out_specs=[
            pl.BlockSpec((2, RB, F_H), lambda i: (0, i, 0)),
            pl.BlockSpec((RB, F_H), lambda i: (i, 0)),
        ],
        out_shape=[
            jax.ShapeDtypeStruct((2, NPAD, F_H), jnp.float32),
            jax.ShapeDtypeStruct((NPAD, F_H), jnp.float32),
        ],
    )(xp, W1, b1.reshape(1, F_H), Wd1, dinv1)


def _layer2_body(so_ref, sn_ref, tn_ref, d_ref, o_ref, w2_ref, b2_ref,
                 wd2_ref, a1_ref, a3_ref, tab2_ref, t2_ref, y2_ref):
    d = d_ref[...]
    o = o_ref[...]
    h1 = _prelu(d * (sn_ref[...] + tn_ref[...]), a1_ref[0, 0])
    hn = _prelu(o * so_ref[...], a3_ref[0, 0])
    hw2 = _mm(h1, w2_ref[...]) + b2_ref[...]
    t2 = hw2 * d
    tab2_ref[0] = t2[:, :F_H // 2]
    tab2_ref[1] = t2[:, F_H // 2:]
    t2_ref[...] = t2
    y2_ref[...] = _mm(hn, wd2_ref[...])


def _tc_layer2(s_orig, s_norm, t_nrm, dinv1, oinv1, W2, b2, Wd2, a1, a3):
    full = lambda r, c: pl.BlockSpec((r, c), lambda i: (0, 0))
    blk = lambda c: pl.BlockSpec((RB, c), lambda i: (i, 0))
    return pl.pallas_call(
        _layer2_body,
        grid=(NRB,),
        in_specs=[
            blk(F_H), blk(F_H), blk(F_H),
            blk(1), blk(1),
            full(F_H, F_H), full(1, F_H), full(F_H, F_H),
            full(1, 1), full(1, 1),
        ],
        out_specs=[
            pl.BlockSpec((2, RB, F_H // 2), lambda i: (0, i, 0)),
            blk(F_H), blk(F_H),
        ],
        out_shape=[
            jax.ShapeDtypeStruct((2, NPAD, F_H // 2), jnp.float32),
            jax.ShapeDtypeStruct((NPAD, F_H), jnp.float32),
            jax.ShapeDtypeStruct((NPAD, F_H), jnp.float32),
        ],
    )(s_orig, s_norm, t_nrm, dinv1, oinv1, W2, b2.reshape(1, F_H), Wd2,
      a1.reshape(1, 1), a3.reshape(1, 1))


def _layer3_body(sa_ref, sb_ref, t2_ref, d_ref, a2_ref, y1_ref, y2_ref,
                 bd1_ref, bd2_ref, h2_ref, mip_ref, lmip_ref):
    d = d_ref[...]
    s = jnp.concatenate([sa_ref[...], sb_ref[...]], axis=1)
    h2 = _prelu(d * (s + t2_ref[...]), a2_ref[0, 0])
    h2_ref[...] = h2
    mip_ref[...] = jax.nn.sigmoid(
        jnp.sum(y1_ref[...] * h2, axis=-1) + bd1_ref[0, 0]).reshape(RB, 1)
    lmip_ref[...] = jax.nn.sigmoid(
        jnp.sum(y2_ref[...] * h2, axis=-1) + bd2_ref[0, 0]).reshape(RB, 1)


def _tc_layer3(s2a, s2b, t2, dinv1, a2, y1, y2, bd1, bd2):
    full = lambda r, c: pl.BlockSpec((r, c), lambda i: (0, 0))
    blk = lambda c: pl.BlockSpec((RB, c), lambda i: (i, 0))
    return pl.pallas_call(
        _layer3_body,
        grid=(NRB,),
        in_specs=[
            blk(F_H // 2), blk(F_H // 2), blk(F_H),
            blk(1),
            full(1, 1), blk(F_H), blk(F_H), full(1, 1), full(1, 1),
        ],
        out_specs=[
            blk(F_H),
            blk(1),
            blk(1),
        ],
        out_shape=[
            jax.ShapeDtypeStruct((NPAD, F_H), jnp.float32),
            jax.ShapeDtypeStruct((NPAD, 1), jnp.float32),
            jax.ShapeDtypeStruct((NPAD, 1), jnp.float32),
        ],
    )(s2a, s2b, t2, dinv1, a2.reshape(1, 1), y1, y2,
      bd1.reshape(1, 1), bd2.reshape(1, 1))


def _neg_body(g_ref, h2_ref, bd1_ref, bd2_ref, mn_ref, lmn_ref):
    gb = g_ref[0]
    h2 = h2_ref[...]
    mn_ref[...] = jax.nn.sigmoid(
        jnp.sum(gb[:, :F_H] * h2, axis=-1) + bd1_ref[0, 0]).reshape(1, RB, 1)
    lmn_ref[...] = jax.nn.sigmoid(
        jnp.sum(gb[:, F_H:] * h2, axis=-1) + bd2_ref[0, 0]).reshape(1, RB, 1)


def _tc_neg(g, h2, bd1, bd2):
    return pl.pallas_call(
        _neg_body,
        grid=(N_NEG, NRB),
        in_specs=[
            pl.BlockSpec((1, RB, 2 * F_H), lambda k, j: (k, j, 0)),
            pl.BlockSpec((RB, F_H), lambda k, j: (j, 0)),
            pl.BlockSpec((1, 1), lambda k, j: (0, 0)),
            pl.BlockSpec((1, 1), lambda k, j: (0, 0)),
        ],
        out_specs=[
            pl.BlockSpec((1, RB, 1), lambda k, j: (k, j, 0)),
            pl.BlockSpec((1, RB, 1), lambda k, j: (k, j, 0)),
        ],
        out_shape=[
            jax.ShapeDtypeStruct((N_NEG, NPAD, 1), jnp.float32),
            jax.ShapeDtypeStruct((N_NEG, NPAD, 1), jnp.float32),
        ],
    )(g, h2, bd1.reshape(1, 1), bd2.reshape(1, 1))


def _adj_body(a_ref, b_ref, o_ref):
    acc = lax.dot_general(a_ref[...], b_ref[...], (((1,), (1,)), ((), ())),
                          preferred_element_type=jnp.float32)
    o_ref[...] = jax.nn.sigmoid(acc)


def _adj_rebuilt(h2):
    return pl.pallas_call(
        _adj_body,
        grid=(pl.cdiv(N, BM), pl.cdiv(N, BN)),
        in_specs=[
            pl.BlockSpec((BM, F_H), lambda i, j: (i, 0)),
            pl.BlockSpec((BN, F_H), lambda i, j: (j, 0)),
        ],
        out_specs=pl.BlockSpec((BM, BN), lambda i, j: (i, j)),
        out_shape=jax.ShapeDtypeStruct((N, N), jnp.float32),
    )(h2, h2)


def kernel(features_norm, edge_index, neg_sample_list, W1, b1, a1, W2, b2, a2, a3, Wd1, bd1, Wd2, bd2):
    x = jnp.squeeze(features_norm, 0)
    src, dst = edge_index[0], edge_index[1]

    pad = EPAD - E
    srcp = jnp.concatenate([src, jnp.zeros((pad,), jnp.int32)]).reshape(CHT, 64)
    dstp = jnp.concatenate([dst, jnp.full((pad,), TRASH, jnp.int32)]).reshape(CHT, 64)
    srcoff = jnp.stack([srcp, srcp + NPAD])
    dst128 = dstp.reshape(EPAD // CH, CH)

    z16 = jnp.zeros((NPAD, 16), jnp.float32)
    z32 = jnp.zeros((NPAD, F_H // 2), jnp.float32)
    z64 = jnp.zeros((NPAD, F_H), jnp.float32)
    ones16 = jnp.ones((CH, 16), jnp.float32)
    xp = jnp.zeros((NPAD, F_IN), jnp.float32).at[:N].set(x)

    cnt2 = _sc_degree()(ones16, dst128, z16)
    cnt = cnt2[0, :, 0] + cnt2[1, :, 0]
    dinv1 = lax.rsqrt(cnt + 1.0).reshape(NPAD, 1)
    oinv1 = (1.0 / jnp.maximum(cnt, 1.0)).reshape(NPAD, 1)

    tab1, y1 = _tc_layer1(xp, W1, b1, Wd1, dinv1)

    s1 = _sc_spmm_split(F_H, 8)(tab1.reshape(2 * NPAD, F_H), srcoff, dstp, z64)

    tab2, t2, y2 = _tc_layer2(s1[0], s1[1], tab1[1], dinv1, oinv1,
                              W2, b2, Wd2, a1, a3)

    s2 = _sc_spmm_split(F_H // 2, 8)(tab2.reshape(2 * NPAD, F_H // 2), srcoff, dstp, z32)

    h2, mip, lmip = _tc_layer3(s2[0], s2[1], t2, dinv1, a2, y1, y2, bd1, bd2)

    y12 = jnp.concatenate([y1, y2], axis=1)
    negp = jnp.pad(neg_sample_list, ((0, 0), (0, NPAD - N))).reshape(NW, GCPW, GCH)
    g = _sc_gather_rows(2 * F_H)(y12, negp).reshape(N_NEG, NPAD, 2 * F_H)

    mn, lmn = _tc_neg(g, h2, bd1, bd2)
    adj_rebuilt = _adj_rebuilt(h2[:N])

    mi_pos = mip.reshape(NPAD)[:N]
    local_mi_pos = lmip.reshape(NPAD)[:N]
    mi_neg = mn.reshape(N_NEG, NPAD)[:, :N]
    local_mi_neg = lmn.reshape(N_NEG, NPAD)[:, :N]
    return (mi_pos, mi_neg, local_mi_pos, local_mi_neg, adj_rebuilt)


# speedup vs baseline: 8.6403x; 1.1426x over previous
import functools

import jax
import jax.numpy as jnp
from jax import lax
from jax.experimental import pallas as pl
from jax.experimental.pallas import tpu as pltpu
from jax.experimental.pallas import tpu_sc as plsc

N = 10000
E = 160000
F_IN = 128
F_H = 64
N_NEG = 5

NC, NS = 2, 16
NW = NC * NS
NPAD = 10240
TRASH = N + 100
CH = 128
EPAD = 163840
CPW = EPAD // (NW * CH)
ROWS_PT = NPAD // NS

CHT = EPAD // 64
CPS = CHT // NS
HALF = CPS // 2

RB = 2048
NRB = NPAD // RB
BM, BN = 1024, 2048


def _prelu(x, a):
    return jnp.where(x >= 0, x, a * x)


def _sc_spmm_split(D, nb):
    mesh = plsc.VectorSubcoreMesh(core_axis_name="c", subcore_axis_name="s")

    @functools.partial(
        pl.kernel,
        out_type=jax.ShapeDtypeStruct((NC, NPAD, D), jnp.float32),
        mesh=mesh,
        scratch_types=[
            pltpu.VMEM((HALF, 64), jnp.int32),
            pltpu.VMEM((HALF, 64), jnp.int32),
            pltpu.VMEM((nb, 64, D), jnp.float32),
            pltpu.VMEM_SHARED((NPAD, D), jnp.float32),
            pltpu.SemaphoreType.DMA,
            pltpu.SemaphoreType.DMA,
        ],
        compiler_params=pltpu.CompilerParams(use_tc_tiling_on_sc=False),
    )
    def k(table_hbm, srcoff_hbm, dst_hbm, zeros_hbm, out_hbm, srcv, dstv, rows, acc, gsem, ssem):
        cid = lax.axis_index("c")
        sid = lax.axis_index("s")
        r0 = sid * ROWS_PT
        pltpu.sync_copy(zeros_hbm.at[pl.ds(r0, ROWS_PT)], acc.at[pl.ds(r0, ROWS_PT)])
        plsc.subcore_barrier()

        c_base = sid * CPS
        for h in range(2):
            pltpu.sync_copy(srcoff_hbm.at[cid, pl.ds(c_base + h * HALF, HALF)], srcv)
            pltpu.sync_copy(dst_hbm.at[pl.ds(c_base + h * HALF, HALF)], dstv)

            def group(g, _):
                c0 = g * nb
                gds = [pltpu.async_copy(table_hbm.at[srcv.at[c0 + b]], rows.at[b], gsem)
                       for b in range(nb)]
                for gd in gds:
                    gd.wait()
                sds = [pltpu.async_copy(rows.at[b], acc.at[dstv.at[c0 + b]], ssem, add=True)
                       for b in range(nb)]
                for sd in sds:
                    sd.wait()
                return 0

            lax.fori_loop(0, HALF // nb, group, 0)
        plsc.subcore_barrier()
        pltpu.sync_copy(acc.at[pl.ds(r0, ROWS_PT)], out_hbm.at[cid, pl.ds(r0, ROWS_PT)])

    return k


def _sc_degree():
    mesh = plsc.VectorSubcoreMesh(core_axis_name="c", subcore_axis_name="s")

    @functools.partial(
        pl.kernel,
        out_type=jax.ShapeDtypeStruct((NC, NPAD, 16), jnp.float32),
        mesh=mesh,
        scratch_types=[
            pltpu.VMEM((CPW, CH), jnp.int32),
            pltpu.VMEM((CH, 16), jnp.float32),
            pltpu.VMEM_SHARED((NPAD, 16), jnp.float32),
            pltpu.SemaphoreType.DMA,
        ],
        compiler_params=pltpu.CompilerParams(use_tc_tiling_on_sc=False),
    )
    def k(ones_hbm, dst_hbm, zeros_hbm, out_hbm, dstv, ones_v, acc, ssem):
        cid = lax.axis_index("c")
        sid = lax.axis_index("s")
        wid = sid * NC + cid
        r0 = sid * ROWS_PT
        pltpu.sync_copy(zeros_hbm.at[pl.ds(r0, ROWS_PT)], acc.at[pl.ds(r0, ROWS_PT)])
        pltpu.sync_copy(ones_hbm, ones_v)
        pltpu.sync_copy(dst_hbm.at[pl.ds(wid * CPW, CPW)], dstv)
        plsc.subcore_barrier()

        def group(g, _):
            sds = [pltpu.async_copy(ones_v, acc.at[dstv.at[g * 8 + b]], ssem, add=True)
                   for b in range(8)]
            for sd in sds:
                sd.wait()
            return 0

        lax.fori_loop(0, CPW // 8, group, 0)
        plsc.subcore_barrier()
        pltpu.sync_copy(acc.at[pl.ds(r0, ROWS_PT)], out_hbm.at[cid, pl.ds(r0, ROWS_PT)])

    return k


GCH = 80
GTOT = N_NEG * NPAD
GCPW = GTOT // (NW * GCH)
GNB = 5


def _sc_gather_rows(D):
    mesh = plsc.VectorSubcoreMesh(core_axis_name="c", subcore_axis_name="s")

    @functools.partial(
        pl.kernel,
        out_type=jax.ShapeDtypeStruct((GTOT, D), jnp.float32),
        mesh=mesh,
        scratch_types=[
            pltpu.VMEM((GCPW, GCH), jnp.int32),
            pltpu.VMEM((GNB, GCH, D), jnp.float32),
            pltpu.SemaphoreType.DMA,
            pltpu.SemaphoreType.DMA,
        ],
    )
    def k(table_hbm, idx_hbm, out_hbm, idxv, rows, gsem, ssem):
        cid = lax.axis_index("c")
        sid = lax.axis_index("s")
        wid = sid * NC + cid
        pltpu.sync_copy(idx_hbm.at[wid], idxv)

        def group(g, _):
            c0 = g * GNB
            gds = [pltpu.async_copy(table_hbm.at[idxv.at[c0 + b]], rows.at[b], gsem)
                   for b in range(GNB)]
            for gd in gds:
                gd.wait()
            sds = [pltpu.async_copy(rows.at[b],
                                    out_hbm.at[pl.ds((wid * GCPW + c0 + b) * GCH, GCH)],
                                    ssem)
                   for b in range(GNB)]
            for sd in sds:
                sd.wait()
            return 0

        lax.fori_loop(0, GCPW // GNB, group, 0)

    return k


def _mm(a, b):
    return lax.dot_general(a, b, (((1,), (0,)), ((), ())),
                           preferred_element_type=jnp.float32)


def _layer1_body(x_ref, w1_ref, b1_ref, wd1_ref, d_ref, tab_ref, y1_ref):
    xb = x_ref[...]
    hw1 = _mm(xb, w1_ref[...]) + b1_ref[...]
    d = d_ref[...]
    tab_ref[0] = hw1
    tab_ref[1] = hw1 * d
    y1_ref[...] = _mm(xb, wd1_ref[...])


def _tc_layer1(xp, W1, b1, Wd1, dinv1):
    return pl.pallas_call(
        _layer1_body,
        grid=(NRB,),
        in_specs=[
            pl.BlockSpec((RB, F_IN), lambda i: (i, 0)),
            pl.BlockSpec((F_IN, F_H), lambda i: (0, 0)),
            pl.BlockSpec((1, F_H), lambda i: (0, 0)),
            pl.BlockSpec((F_IN, F_H), lambda i: (0, 0)),
            pl.BlockSpec((RB, 1), lambda i: (i, 0)),
        ],
        out_specs=[
            pl.BlockSpec((2, RB, F_H), lambda i: (0, i, 0)),
            pl.BlockSpec((RB, F_H), lambda i: (i, 0)),
        ],
        out_shape=[
            jax.ShapeDtypeStruct((2, NPAD, F_H), jnp.float32),
            jax.ShapeDtypeStruct((NPAD, F_H), jnp.float32),
        ],
    )(xp, W1, b1.reshape(1, F_H), Wd1, dinv1)


def _layer2_body(so_ref, sn_ref, tn_ref, d_ref, o_ref, w2_ref, b2_ref,
                 wd2_ref, a1_ref, a3_ref, tab2_ref, t2_ref, y2_ref):
    d = d_ref[...]
    o = o_ref[...]
    h1 = _prelu(d * (sn_ref[...] + tn_ref[...]), a1_ref[0, 0])
    hn = _prelu(o * so_ref[...], a3_ref[0, 0])
    hw2 = _mm(h1, w2_ref[...]) + b2_ref[...]
    t2 = hw2 * d
    tab2_ref[0] = t2[:, :F_H // 2]
    tab2_ref[1] = t2[:, F_H // 2:]
    t2_ref[...] = t2
    y2_ref[...] = _mm(hn, wd2_ref[...])


def _tc_layer2(s_orig, s_norm, t_nrm, dinv1, oinv1, W2, b2, Wd2, a1, a3):
    full = lambda r, c: pl.BlockSpec((r, c), lambda i: (0, 0))
    blk = lambda c: pl.BlockSpec((RB, c), lambda i: (i, 0))
    return pl.pallas_call(
        _layer2_body,
        grid=(NRB,),
        in_specs=[
            blk(F_H), blk(F_H), blk(F_H),
            blk(1), blk(1),
            full(F_H, F_H), full(1, F_H), full(F_H, F_H),
            full(1, 1), full(1, 1),
        ],
        out_specs=[
            pl.BlockSpec((2, RB, F_H // 2), lambda i: (0, i, 0)),
            blk(F_H), blk(F_H),
        ],
        out_shape=[
            jax.ShapeDtypeStruct((2, NPAD, F_H // 2), jnp.float32),
            jax.ShapeDtypeStruct((NPAD, F_H), jnp.float32),
            jax.ShapeDtypeStruct((NPAD, F_H), jnp.float32),
        ],
    )(s_orig, s_norm, t_nrm, dinv1, oinv1, W2, b2.reshape(1, F_H), Wd2,
      a1.reshape(1, 1), a3.reshape(1, 1))


def _layer3_body(sa_ref, sb_ref, t2_ref, d_ref, a2_ref, y1_ref, y2_ref,
                 bd1_ref, bd2_ref, h2_ref, mip_ref, lmip_ref):
    d = d_ref[...]
    s = jnp.concatenate([sa_ref[...], sb_ref[...]], axis=1)
    h2 = _prelu(d * (s + t2_ref[...]), a2_ref[0, 0])
    h2_ref[...] = h2
    mip_ref[...] = jax.nn.sigmoid(
        jnp.sum(y1_ref[...] * h2, axis=-1) + bd1_ref[0, 0]).reshape(RB, 1)
    lmip_ref[...] = jax.nn.sigmoid(
        jnp.sum(y2_ref[...] * h2, axis=-1) + bd2_ref[0, 0]).reshape(RB, 1)


def _tc_layer3(s2a, s2b, t2, dinv1, a2, y1, y2, bd1, bd2):
    full = lambda r, c: pl.BlockSpec((r, c), lambda i: (0, 0))
    blk = lambda c: pl.BlockSpec((RB, c), lambda i: (i, 0))
    return pl.pallas_call(
        _layer3_body,
        grid=(NRB,),
        in_specs=[
            blk(F_H // 2), blk(F_H // 2), blk(F_H),
            blk(1),
            full(1, 1), blk(F_H), blk(F_H), full(1, 1), full(1, 1),
        ],
        out_specs=[
            blk(F_H),
            blk(1),
            blk(1),
        ],
        out_shape=[
            jax.ShapeDtypeStruct((NPAD, F_H), jnp.float32),
            jax.ShapeDtypeStruct((NPAD, 1), jnp.float32),
            jax.ShapeDtypeStruct((NPAD, 1), jnp.float32),
        ],
    )(s2a, s2b, t2, dinv1, a2.reshape(1, 1), y1, y2,
      bd1.reshape(1, 1), bd2.reshape(1, 1))


def _neg_body(g_ref, h2_ref, bd1_ref, bd2_ref, mn_ref, lmn_ref):
    gb = g_ref[0]
    h2 = h2_ref[...]
    mn_ref[...] = jax.nn.sigmoid(
        jnp.sum(gb[:, :F_H] * h2, axis=-1) + bd1_ref[0, 0]).reshape(1, RB, 1)
    lmn_ref[...] = jax.nn.sigmoid(
        jnp.sum(gb[:, F_H:] * h2, axis=-1) + bd2_ref[0, 0]).reshape(1, RB, 1)


def _tc_neg(g, h2, bd1, bd2):
    return pl.pallas_call(
        _neg_body,
        grid=(N_NEG, NRB),
        in_specs=[
            pl.BlockSpec((1, RB, 2 * F_H), lambda k, j: (k, j, 0)),
            pl.BlockSpec((RB, F_H), lambda k, j: (j, 0)),
            pl.BlockSpec((1, 1), lambda k, j: (0, 0)),
            pl.BlockSpec((1, 1), lambda k, j: (0, 0)),
        ],
        out_specs=[
            pl.BlockSpec((1, RB, 1), lambda k, j: (k, j, 0)),
            pl.BlockSpec((1, RB, 1), lambda k, j: (k, j, 0)),
        ],
        out_shape=[
            jax.ShapeDtypeStruct((N_NEG, NPAD, 1), jnp.float32),
            jax.ShapeDtypeStruct((N_NEG, NPAD, 1), jnp.float32),
        ],
    )(g, h2, bd1.reshape(1, 1), bd2.reshape(1, 1))


def _adj_body(a_ref, b_ref, o_ref):
    acc = lax.dot_general(a_ref[...], b_ref[...], (((1,), (1,)), ((), ())),
                          preferred_element_type=jnp.float32)
    o_ref[...] = jax.nn.sigmoid(acc)


def _adj_rebuilt(h2):
    return pl.pallas_call(
        _adj_body,
        grid=(pl.cdiv(N, BM), pl.cdiv(N, BN)),
        in_specs=[
            pl.BlockSpec((BM, F_H), lambda i, j: (i, 0)),
            pl.BlockSpec((BN, F_H), lambda i, j: (j, 0)),
        ],
        out_specs=pl.BlockSpec((BM, BN), lambda i, j: (i, j)),
        out_shape=jax.ShapeDtypeStruct((N, N), jnp.float32),
    )(h2, h2)


def kernel(features_norm, edge_index, neg_sample_list, W1, b1, a1, W2, b2, a2, a3, Wd1, bd1, Wd2, bd2):
    x = jnp.squeeze(features_norm, 0)
    src, dst = edge_index[0], edge_index[1]

    pad = EPAD - E
    srcp = jnp.concatenate([src, jnp.zeros((pad,), jnp.int32)]).reshape(CHT, 64)
    dstp = jnp.concatenate([dst, jnp.full((pad,), TRASH, jnp.int32)]).reshape(CHT, 64)
    srcoff = jnp.stack([srcp, srcp + NPAD])
    dst128 = dstp.reshape(EPAD // CH, CH)

    z16 = jnp.zeros((NPAD, 16), jnp.float32)
    z32 = jnp.zeros((NPAD, F_H // 2), jnp.float32)
    z64 = jnp.zeros((NPAD, F_H), jnp.float32)
    ones16 = jnp.ones((CH, 16), jnp.float32)
    xp = jnp.zeros((NPAD, F_IN), jnp.float32).at[:N].set(x)

    cnt2 = _sc_degree()(ones16, dst128, z16)
    cnt = cnt2[0, :, 0] + cnt2[1, :, 0]
    dinv1 = lax.rsqrt(cnt + 1.0).reshape(NPAD, 1)
    oinv1 = (1.0 / jnp.maximum(cnt, 1.0)).reshape(NPAD, 1)

    tab1, y1 = _tc_layer1(xp, W1, b1, Wd1, dinv1)

    s1 = _sc_spmm_split(F_H, 8)(tab1.reshape(2 * NPAD, F_H), srcoff, dstp, z64)

    tab2, t2, y2 = _tc_layer2(s1[0], s1[1], tab1[1], dinv1, oinv1,
                              W2, b2, Wd2, a1, a3)

    s2 = _sc_spmm_split(F_H // 2, 8)(tab2.reshape(2 * NPAD, F_H // 2), srcoff, dstp, z32)

    h2, mip, lmip = _tc_layer3(s2[0], s2[1], t2, dinv1, a2, y1, y2, bd1, bd2)

    y12 = jnp.concatenate([y1, y2], axis=1)
    negp = jnp.pad(neg_sample_list, ((0, 0), (0, NPAD - N))).reshape(NW, GCPW, GCH)
    g = _sc_gather_rows(2 * F_H)(y12, negp).reshape(N_NEG, NPAD, 2 * F_H)

    adj_rebuilt = _adj_rebuilt(h2[:N])
    mn, lmn = _tc_neg(g, h2, bd1, bd2)

    mi_pos = mip.reshape(NPAD)[:N]
    local_mi_pos = lmip.reshape(NPAD)[:N]
    mi_neg = mn.reshape(N_NEG, NPAD)[:, :N]
    local_mi_neg = lmn.reshape(N_NEG, NPAD)[:, :N]
    return (mi_pos, mi_neg, local_mi_pos, local_mi_neg, adj_rebuilt)


# speedup vs baseline: 9.1920x; 1.0639x over previous
import functools

import jax
import jax.numpy as jnp
from jax import lax
from jax.experimental import pallas as pl
from jax.experimental.pallas import tpu as pltpu
from jax.experimental.pallas import tpu_sc as plsc

N = 10000
E = 160000
F_IN = 128
F_H = 64
N_NEG = 5

NC, NS = 2, 16
NW = NC * NS
NPAD = 10240
TRASH = N + 100
CH = 128
EPAD = 163840
CPW = EPAD // (NW * CH)
ROWS_PT = NPAD // NS

CHT = EPAD // 64
CPS = CHT // NS
HALF = CPS // 2

RB = 2048
NRB = NPAD // RB
BM, BN = 1024, 2048


def _prelu(x, a):
    return jnp.where(x >= 0, x, a * x)


def _sc_spmm_split(D, nb):
    mesh = plsc.VectorSubcoreMesh(core_axis_name="c", subcore_axis_name="s")

    @functools.partial(
        pl.kernel,
        out_type=jax.ShapeDtypeStruct((NC, NPAD, D), jnp.float32),
        mesh=mesh,
        scratch_types=[
            pltpu.VMEM((HALF, 64), jnp.int32),
            pltpu.VMEM((HALF, 64), jnp.int32),
            pltpu.VMEM((nb, 64, D), jnp.float32),
            pltpu.VMEM_SHARED((NPAD, D), jnp.float32),
            pltpu.SemaphoreType.DMA,
            pltpu.SemaphoreType.DMA,
        ],
        compiler_params=pltpu.CompilerParams(use_tc_tiling_on_sc=False),
    )
    def k(table_hbm, srcoff_hbm, dst_hbm, zeros_hbm, out_hbm, srcv, dstv, rows, acc, gsem, ssem):
        cid = lax.axis_index("c")
        sid = lax.axis_index("s")
        r0 = sid * ROWS_PT
        pltpu.sync_copy(zeros_hbm.at[pl.ds(r0, ROWS_PT)], acc.at[pl.ds(r0, ROWS_PT)])
        plsc.subcore_barrier()

        c_base = sid * CPS
        for h in range(2):
            pltpu.sync_copy(srcoff_hbm.at[cid, pl.ds(c_base + h * HALF, HALF)], srcv)
            pltpu.sync_copy(dst_hbm.at[pl.ds(c_base + h * HALF, HALF)], dstv)

            def group(g, _):
                c0 = g * nb
                gds = [pltpu.async_copy(table_hbm.at[srcv.at[c0 + b]], rows.at[b], gsem)
                       for b in range(nb)]
                for gd in gds:
                    gd.wait()
                sds = [pltpu.async_copy(rows.at[b], acc.at[dstv.at[c0 + b]], ssem, add=True)
                       for b in range(nb)]
                for sd in sds:
                    sd.wait()
                return 0

            lax.fori_loop(0, HALF // nb, group, 0)
        plsc.subcore_barrier()
        pltpu.sync_copy(acc.at[pl.ds(r0, ROWS_PT)], out_hbm.at[cid, pl.ds(r0, ROWS_PT)])

    return k


def _sc_degree():
    mesh = plsc.VectorSubcoreMesh(core_axis_name="c", subcore_axis_name="s")

    @functools.partial(
        pl.kernel,
        out_type=jax.ShapeDtypeStruct((NC, NPAD, 16), jnp.float32),
        mesh=mesh,
        scratch_types=[
            pltpu.VMEM((CPW, CH), jnp.int32),
            pltpu.VMEM((CH, 16), jnp.float32),
            pltpu.VMEM_SHARED((NPAD, 16), jnp.float32),
            pltpu.SemaphoreType.DMA,
        ],
        compiler_params=pltpu.CompilerParams(use_tc_tiling_on_sc=False),
    )
    def k(ones_hbm, dst_hbm, zeros_hbm, out_hbm, dstv, ones_v, acc, ssem):
        cid = lax.axis_index("c")
        sid = lax.axis_index("s")
        wid = sid * NC + cid
        r0 = sid * ROWS_PT
        pltpu.sync_copy(zeros_hbm.at[pl.ds(r0, ROWS_PT)], acc.at[pl.ds(r0, ROWS_PT)])
        pltpu.sync_copy(ones_hbm, ones_v)
        pltpu.sync_copy(dst_hbm.at[pl.ds(wid * CPW, CPW)], dstv)
        plsc.subcore_barrier()

        def group(g, _):
            sds = [pltpu.async_copy(ones_v, acc.at[dstv.at[g * 8 + b]], ssem, add=True)
                   for b in range(8)]
            for sd in sds:
                sd.wait()
            return 0

        lax.fori_loop(0, CPW // 8, group, 0)
        plsc.subcore_barrier()
        pltpu.sync_copy(acc.at[pl.ds(r0, ROWS_PT)], out_hbm.at[cid, pl.ds(r0, ROWS_PT)])

    return k


GCH = 80
GTOT = N_NEG * NPAD
GCPW = GTOT // (NW * GCH)
GNB = 5


def _sc_gather_rows(D):
    mesh = plsc.VectorSubcoreMesh(core_axis_name="c", subcore_axis_name="s")

    @functools.partial(
        pl.kernel,
        out_type=jax.ShapeDtypeStruct((GTOT, D), jnp.float32),
        mesh=mesh,
        scratch_types=[
            pltpu.VMEM((GCPW, GCH), jnp.int32),
            pltpu.VMEM((GNB, GCH, D), jnp.float32),
            pltpu.SemaphoreType.DMA,
            pltpu.SemaphoreType.DMA,
        ],
    )
    def k(table_hbm, idx_hbm, out_hbm, idxv, rows, gsem, ssem):
        cid = lax.axis_index("c")
        sid = lax.axis_index("s")
        wid = sid * NC + cid
        pltpu.sync_copy(idx_hbm.at[wid], idxv)

        def group(g, _):
            c0 = g * GNB
            gds = [pltpu.async_copy(table_hbm.at[idxv.at[c0 + b]], rows.at[b], gsem)
                   for b in range(GNB)]
            for gd in gds:
                gd.wait()
            sds = [pltpu.async_copy(rows.at[b],
                                    out_hbm.at[pl.ds((wid * GCPW + c0 + b) * GCH, GCH)],
                                    ssem)
                   for b in range(GNB)]
            for sd in sds:
                sd.wait()
            return 0

        lax.fori_loop(0, GCPW // GNB, group, 0)

    return k


def _mm(a, b):
    return lax.dot_general(a, b, (((1,), (0,)), ((), ())),
                           preferred_element_type=jnp.float32)


def _rowvec(r_ref):
    return r_ref[...].reshape(RB, 1)


def _layer1_body(x_ref, w1_ref, b1_ref, wd1_ref, d_ref, tab_ref, y1_ref):
    xb = x_ref[...]
    hw1 = _mm(xb, w1_ref[...]) + b1_ref[...]
    d = _rowvec(d_ref)
    tab_ref[0] = hw1
    tab_ref[1] = hw1 * d
    y1_ref[...] = _mm(xb, wd1_ref[...])


def _tc_layer1(xp, W1, b1, Wd1, dinvr):
    return pl.pallas_call(
        _layer1_body,
        grid=(NRB,),
        in_specs=[
            pl.BlockSpec((RB, F_IN), lambda i: (i, 0)),
            pl.BlockSpec((F_IN, F_H), lambda i: (0, 0)),
            pl.BlockSpec((1, F_H), lambda i: (0, 0)),
            pl.BlockSpec((F_IN, F_H), lambda i: (0, 0)),
            pl.BlockSpec((1, RB), lambda i: (0, i)),
        ],
        out_specs=[
            pl.BlockSpec((2, RB, F_H), lambda i: (0, i, 0)),
            pl.BlockSpec((RB, F_H), lambda i: (i, 0)),
        ],
        out_shape=[
            jax.ShapeDtypeStruct((2, NPAD, F_H), jnp.float32),
            jax.ShapeDtypeStruct((NPAD, F_H), jnp.float32),
        ],
    )(xp, W1, b1.reshape(1, F_H), Wd1, dinvr)


def _layer2_body(so_ref, sn_ref, tn_ref, y1_ref, d_ref, o_ref, w2_ref, b2_ref,
                 wd2_ref, a1_ref, a3_ref, tab2_ref, t2_ref, y12_ref):
    d = _rowvec(d_ref)
    o = _rowvec(o_ref)
    h1 = _prelu(d * (sn_ref[...] + tn_ref[...]), a1_ref[0, 0])
    hn = _prelu(o * so_ref[...], a3_ref[0, 0])
    hw2 = _mm(h1, w2_ref[...]) + b2_ref[...]
    t2 = hw2 * d
    tab2_ref[0] = t2[:, :F_H // 2]
    tab2_ref[1] = t2[:, F_H // 2:]
    t2_ref[...] = t2
    y12_ref[...] = jnp.concatenate(
        [y1_ref[...], _mm(hn, wd2_ref[...])], axis=1)


def _tc_layer2(s_orig, s_norm, t_nrm, y1, dinvr, oinvr, W2, b2, Wd2, a1, a3):
    full = lambda r, c: pl.BlockSpec((r, c), lambda i: (0, 0))
    blk = lambda c: pl.BlockSpec((RB, c), lambda i: (i, 0))
    row = pl.BlockSpec((1, RB), lambda i: (0, i))
    return pl.pallas_call(
        _layer2_body,
        grid=(NRB,),
        in_specs=[
            blk(F_H), blk(F_H), blk(F_H), blk(F_H),
            row, row,
            full(F_H, F_H), full(1, F_H), full(F_H, F_H),
            full(1, 1), full(1, 1),
        ],
        out_specs=[
            pl.BlockSpec((2, RB, F_H // 2), lambda i: (0, i, 0)),
            blk(F_H), blk(2 * F_H),
        ],
        out_shape=[
            jax.ShapeDtypeStruct((2, NPAD, F_H // 2), jnp.float32),
            jax.ShapeDtypeStruct((NPAD, F_H), jnp.float32),
            jax.ShapeDtypeStruct((NPAD, 2 * F_H), jnp.float32),
        ],
    )(s_orig, s_norm, t_nrm, y1, dinvr, oinvr, W2, b2.reshape(1, F_H), Wd2,
      a1.reshape(1, 1), a3.reshape(1, 1))


def _layer3_body(sa_ref, sb_ref, t2_ref, d_ref, a2_ref, y12_ref,
                 bd1_ref, bd2_ref, h2_ref, mip_ref, lmip_ref):
    d = _rowvec(d_ref)
    s = jnp.concatenate([sa_ref[...], sb_ref[...]], axis=1)
    h2 = _prelu(d * (s + t2_ref[...]), a2_ref[0, 0])
    h2_ref[...] = h2
    y12 = y12_ref[...]
    mip_ref[...] = jax.nn.sigmoid(
        jnp.sum(y12[:, :F_H] * h2, axis=-1) + bd1_ref[0, 0]).reshape(1, RB)
    lmip_ref[...] = jax.nn.sigmoid(
        jnp.sum(y12[:, F_H:] * h2, axis=-1) + bd2_ref[0, 0]).reshape(1, RB)


def _tc_layer3(s2a, s2b, t2, dinvr, a2, y12, bd1, bd2):
    full = lambda r, c: pl.BlockSpec((r, c), lambda i: (0, 0))
    blk = lambda c: pl.BlockSpec((RB, c), lambda i: (i, 0))
    row = pl.BlockSpec((1, RB), lambda i: (0, i))
    return pl.pallas_call(
        _layer3_body,
        grid=(NRB,),
        in_specs=[
            blk(F_H // 2), blk(F_H // 2), blk(F_H),
            row,
            full(1, 1), blk(2 * F_H), full(1, 1), full(1, 1),
        ],
        out_specs=[
            blk(F_H),
            row,
            row,
        ],
        out_shape=[
            jax.ShapeDtypeStruct((NPAD, F_H), jnp.float32),
            jax.ShapeDtypeStruct((1, NPAD), jnp.float32),
            jax.ShapeDtypeStruct((1, NPAD), jnp.float32),
        ],
    )(s2a, s2b, t2, dinvr, a2.reshape(1, 1), y12,
      bd1.reshape(1, 1), bd2.reshape(1, 1))


def _neg_body(g_ref, h2_ref, bd1_ref, bd2_ref, mn_ref, lmn_ref):
    gb = g_ref[0]
    h2 = h2_ref[...]
    mn_ref[...] = jax.nn.sigmoid(
        jnp.sum(gb[:, :F_H] * h2, axis=-1) + bd1_ref[0, 0]).reshape(1, 1, RB)
    lmn_ref[...] = jax.nn.sigmoid(
        jnp.sum(gb[:, F_H:] * h2, axis=-1) + bd2_ref[0, 0]).reshape(1, 1, RB)


def _tc_neg(g, h2, bd1, bd2):
    return pl.pallas_call(
        _neg_body,
        grid=(N_NEG, NRB),
        in_specs=[
            pl.BlockSpec((1, RB, 2 * F_H), lambda k, j: (k, j, 0)),
            pl.BlockSpec((RB, F_H), lambda k, j: (j, 0)),
            pl.BlockSpec((1, 1), lambda k, j: (0, 0)),
            pl.BlockSpec((1, 1), lambda k, j: (0, 0)),
        ],
        out_specs=[
            pl.BlockSpec((1, 1, RB), lambda k, j: (k, 0, j)),
            pl.BlockSpec((1, 1, RB), lambda k, j: (k, 0, j)),
        ],
        out_shape=[
            jax.ShapeDtypeStruct((N_NEG, 1, NPAD), jnp.float32),
            jax.ShapeDtypeStruct((N_NEG, 1, NPAD), jnp.float32),
        ],
    )(g, h2, bd1.reshape(1, 1), bd2.reshape(1, 1))


def _adj_body(a_ref, b_ref, o_ref):
    acc = lax.dot_general(a_ref[...], b_ref[...], (((1,), (1,)), ((), ())),
                          preferred_element_type=jnp.float32)
    o_ref[...] = jax.nn.sigmoid(acc)


def _adj_rebuilt(h2):
    return pl.pallas_call(
        _adj_body,
        grid=(pl.cdiv(N, BM), pl.cdiv(N, BN)),
        in_specs=[
            pl.BlockSpec((BM, F_H), lambda i, j: (i, 0)),
            pl.BlockSpec((BN, F_H), lambda i, j: (j, 0)),
        ],
        out_specs=pl.BlockSpec((BM, BN), lambda i, j: (i, j)),
        out_shape=jax.ShapeDtypeStruct((N, N), jnp.float32),
    )(h2, h2)


def kernel(features_norm, edge_index, neg_sample_list, W1, b1, a1, W2, b2, a2, a3, Wd1, bd1, Wd2, bd2):
    x = jnp.squeeze(features_norm, 0)
    src, dst = edge_index[0], edge_index[1]

    pad = EPAD - E
    srcp = jnp.concatenate([src, jnp.zeros((pad,), jnp.int32)]).reshape(CHT, 64)
    dstp = jnp.concatenate([dst, jnp.full((pad,), TRASH, jnp.int32)]).reshape(CHT, 64)
    srcoff = jnp.stack([srcp, srcp + NPAD])
    dst128 = dstp.reshape(EPAD // CH, CH)

    z16 = jnp.zeros((NPAD, 16), jnp.float32)
    z32 = jnp.zeros((NPAD, F_H // 2), jnp.float32)
    z64 = jnp.zeros((NPAD, F_H), jnp.float32)
    ones16 = jnp.ones((CH, 16), jnp.float32)
    xp = jnp.zeros((NPAD, F_IN), jnp.float32).at[:N].set(x)

    cnt2 = _sc_degree()(ones16, dst128, z16)
    cnt = cnt2[0, :, 0] + cnt2[1, :, 0]
    dinvr = lax.rsqrt(cnt + 1.0).reshape(1, NPAD)
    oinvr = (1.0 / jnp.maximum(cnt, 1.0)).reshape(1, NPAD)

    tab1, y1 = _tc_layer1(xp, W1, b1, Wd1, dinvr)

    s1 = _sc_spmm_split(F_H, 8)(tab1.reshape(2 * NPAD, F_H), srcoff, dstp, z64)

    tab2, t2, y12 = _tc_layer2(s1[0], s1[1], tab1[1], y1, dinvr, oinvr,
                               W2, b2, Wd2, a1, a3)

    s2 = _sc_spmm_split(F_H // 2, 8)(tab2.reshape(2 * NPAD, F_H // 2), srcoff, dstp, z32)

    h2, mip, lmip = _tc_layer3(s2[0], s2[1], t2, dinvr, a2, y12, bd1, bd2)

    negp = jnp.pad(neg_sample_list, ((0, 0), (0, NPAD - N))).reshape(NW, GCPW, GCH)
    g = _sc_gather_rows(2 * F_H)(y12, negp).reshape(N_NEG, NPAD, 2 * F_H)

    adj_rebuilt = _adj_rebuilt(h2[:N])
    mn, lmn = _tc_neg(g, h2, bd1, bd2)

    mi_pos = mip[0, :N]
    local_mi_pos = lmip[0, :N]
    mi_neg = mn[:, 0, :N]
    local_mi_neg = lmn[:, 0, :N]
    return (mi_pos, mi_neg, local_mi_pos, local_mi_neg, adj_rebuilt)


# speedup vs baseline: 9.5782x; 1.0420x over previous
import functools

import jax
import jax.numpy as jnp
from jax import lax
from jax.experimental import pallas as pl
from jax.experimental.pallas import tpu as pltpu
from jax.experimental.pallas import tpu_sc as plsc

N = 10000
E = 160000
F_IN = 128
F_H = 64
N_NEG = 5

NC, NS = 2, 16
NW = NC * NS
NPAD = 10240
TRASH = N + 100
CH = 128
EPAD = 163840
CPW = EPAD // (NW * CH)
ROWS_PT = NPAD // NS

CHT = EPAD // 64
CPS = CHT // NS
HALF = CPS // 2

RB = 2048
NRB = NPAD // RB
BM, BN = 1024, 2048


def _prelu(x, a):
    return jnp.where(x >= 0, x, a * x)


def _sc_spmm_split(D, nb):
    mesh = plsc.VectorSubcoreMesh(core_axis_name="c", subcore_axis_name="s")

    @functools.partial(
        pl.kernel,
        out_type=jax.ShapeDtypeStruct((NC, NPAD, D), jnp.float32),
        mesh=mesh,
        scratch_types=[
            pltpu.VMEM((HALF, 64), jnp.int32),
            pltpu.VMEM((HALF, 64), jnp.int32),
            pltpu.VMEM((nb, 64, D), jnp.float32),
            pltpu.VMEM_SHARED((NPAD, D), jnp.float32),
            pltpu.SemaphoreType.DMA,
            pltpu.SemaphoreType.DMA,
        ],
        compiler_params=pltpu.CompilerParams(use_tc_tiling_on_sc=False),
    )
    def k(table_hbm, srcoff_hbm, dst_hbm, zeros_hbm, out_hbm, srcv, dstv, rows, acc, gsem, ssem):
        cid = lax.axis_index("c")
        sid = lax.axis_index("s")
        r0 = sid * ROWS_PT
        pltpu.sync_copy(zeros_hbm.at[pl.ds(r0, ROWS_PT)], acc.at[pl.ds(r0, ROWS_PT)])
        plsc.subcore_barrier()

        c_base = sid * CPS
        for h in range(2):
            pltpu.sync_copy(srcoff_hbm.at[cid, pl.ds(c_base + h * HALF, HALF)], srcv)
            pltpu.sync_copy(dst_hbm.at[pl.ds(c_base + h * HALF, HALF)], dstv)

            def group(g, _):
                c0 = g * nb
                gds = [pltpu.async_copy(table_hbm.at[srcv.at[c0 + b]], rows.at[b], gsem)
                       for b in range(nb)]
                sds = []
                for b in range(nb):
                    gds[b].wait()
                    sds.append(pltpu.async_copy(rows.at[b], acc.at[dstv.at[c0 + b]],
                                                ssem, add=True))
                for sd in sds:
                    sd.wait()
                return 0

            lax.fori_loop(0, HALF // nb, group, 0)
        plsc.subcore_barrier()
        pltpu.sync_copy(acc.at[pl.ds(r0, ROWS_PT)], out_hbm.at[cid, pl.ds(r0, ROWS_PT)])

    return k


def _sc_degree():
    mesh = plsc.VectorSubcoreMesh(core_axis_name="c", subcore_axis_name="s")

    @functools.partial(
        pl.kernel,
        out_type=jax.ShapeDtypeStruct((NC, NPAD, 16), jnp.float32),
        mesh=mesh,
        scratch_types=[
            pltpu.VMEM((CPW, CH), jnp.int32),
            pltpu.VMEM((CH, 16), jnp.float32),
            pltpu.VMEM_SHARED((NPAD, 16), jnp.float32),
            pltpu.SemaphoreType.DMA,
        ],
        compiler_params=pltpu.CompilerParams(use_tc_tiling_on_sc=False),
    )
    def k(ones_hbm, dst_hbm, zeros_hbm, out_hbm, dstv, ones_v, acc, ssem):
        cid = lax.axis_index("c")
        sid = lax.axis_index("s")
        wid = sid * NC + cid
        r0 = sid * ROWS_PT
        pltpu.sync_copy(zeros_hbm.at[pl.ds(r0, ROWS_PT)], acc.at[pl.ds(r0, ROWS_PT)])
        pltpu.sync_copy(ones_hbm, ones_v)
        pltpu.sync_copy(dst_hbm.at[pl.ds(wid * CPW, CPW)], dstv)
        plsc.subcore_barrier()

        def group(g, _):
            sds = [pltpu.async_copy(ones_v, acc.at[dstv.at[g * 8 + b]], ssem, add=True)
                   for b in range(8)]
            for sd in sds:
                sd.wait()
            return 0

        lax.fori_loop(0, CPW // 8, group, 0)
        plsc.subcore_barrier()
        pltpu.sync_copy(acc.at[pl.ds(r0, ROWS_PT)], out_hbm.at[cid, pl.ds(r0, ROWS_PT)])

    return k


GCH = 80
GTOT = N_NEG * NPAD
GCPW = GTOT // (NW * GCH)
GNB = 5


def _sc_gather_rows(D):
    mesh = plsc.VectorSubcoreMesh(core_axis_name="c", subcore_axis_name="s")

    @functools.partial(
        pl.kernel,
        out_type=jax.ShapeDtypeStruct((GTOT, D), jnp.float32),
        mesh=mesh,
        scratch_types=[
            pltpu.VMEM((GCPW, GCH), jnp.int32),
            pltpu.VMEM((GNB, GCH, D), jnp.float32),
            pltpu.SemaphoreType.DMA,
            pltpu.SemaphoreType.DMA,
        ],
    )
    def k(table_hbm, idx_hbm, out_hbm, idxv, rows, gsem, ssem):
        cid = lax.axis_index("c")
        sid = lax.axis_index("s")
        wid = sid * NC + cid
        pltpu.sync_copy(idx_hbm.at[wid], idxv)

        def group(g, _):
            c0 = g * GNB
            gds = [pltpu.async_copy(table_hbm.at[idxv.at[c0 + b]], rows.at[b], gsem)
                   for b in range(GNB)]
            sds = []
            for b in range(GNB):
                gds[b].wait()
                sds.append(pltpu.async_copy(
                    rows.at[b],
                    out_hbm.at[pl.ds((wid * GCPW + c0 + b) * GCH, GCH)],
                    ssem))
            for sd in sds:
                sd.wait()
            return 0

        lax.fori_loop(0, GCPW // GNB, group, 0)

    return k


def _mm(a, b):
    return lax.dot_general(a, b, (((1,), (0,)), ((), ())),
                           preferred_element_type=jnp.float32)


def _rowvec(r_ref):
    return r_ref[...].reshape(RB, 1)


def _layer1_body(x_ref, w1_ref, b1_ref, wd1_ref, d_ref, tab_ref, y1_ref):
    xb = x_ref[...]
    hw1 = _mm(xb, w1_ref[...]) + b1_ref[...]
    d = _rowvec(d_ref)
    tab_ref[0] = hw1
    tab_ref[1] = hw1 * d
    y1_ref[...] = _mm(xb, wd1_ref[...])


def _tc_layer1(xp, W1, b1, Wd1, dinvr):
    return pl.pallas_call(
        _layer1_body,
        grid=(NRB,),
        in_specs=[
            pl.BlockSpec((RB, F_IN), lambda i: (i, 0)),
            pl.BlockSpec((F_IN, F_H), lambda i: (0, 0)),
            pl.BlockSpec((1, F_H), lambda i: (0, 0)),
            pl.BlockSpec((F_IN, F_H), lambda i: (0, 0)),
            pl.BlockSpec((1, RB), lambda i: (0, i)),
        ],
        out_specs=[
            pl.BlockSpec((2, RB, F_H), lambda i: (0, i, 0)),
            pl.BlockSpec((RB, F_H), lambda i: (i, 0)),
        ],
        out_shape=[
            jax.ShapeDtypeStruct((2, NPAD, F_H), jnp.float32),
            jax.ShapeDtypeStruct((NPAD, F_H), jnp.float32),
        ],
    )(xp, W1, b1.reshape(1, F_H), Wd1, dinvr)


def _layer2_body(so_ref, sn_ref, tn_ref, y1_ref, d_ref, o_ref, w2_ref, b2_ref,
                 wd2_ref, a1_ref, a3_ref, tab2_ref, t2_ref, y12_ref):
    d = _rowvec(d_ref)
    o = _rowvec(o_ref)
    h1 = _prelu(d * (sn_ref[...] + tn_ref[...]), a1_ref[0, 0])
    hn = _prelu(o * so_ref[...], a3_ref[0, 0])
    hw2 = _mm(h1, w2_ref[...]) + b2_ref[...]
    t2 = hw2 * d
    tab2_ref[0] = t2[:, :F_H // 2]
    tab2_ref[1] = t2[:, F_H // 2:]
    t2_ref[...] = t2
    y12_ref[...] = jnp.concatenate(
        [y1_ref[...], _mm(hn, wd2_ref[...])], axis=1)


def _tc_layer2(s_orig, s_norm, t_nrm, y1, dinvr, oinvr, W2, b2, Wd2, a1, a3):
    full = lambda r, c: pl.BlockSpec((r, c), lambda i: (0, 0))
    blk = lambda c: pl.BlockSpec((RB, c), lambda i: (i, 0))
    row = pl.BlockSpec((1, RB), lambda i: (0, i))
    return pl.pallas_call(
        _layer2_body,
        grid=(NRB,),
        in_specs=[
            blk(F_H), blk(F_H), blk(F_H), blk(F_H),
            row, row,
            full(F_H, F_H), full(1, F_H), full(F_H, F_H),
            full(1, 1), full(1, 1),
        ],
        out_specs=[
            pl.BlockSpec((2, RB, F_H // 2), lambda i: (0, i, 0)),
            blk(F_H), blk(2 * F_H),
        ],
        out_shape=[
            jax.ShapeDtypeStruct((2, NPAD, F_H // 2), jnp.float32),
            jax.ShapeDtypeStruct((NPAD, F_H), jnp.float32),
            jax.ShapeDtypeStruct((NPAD, 2 * F_H), jnp.float32),
        ],
    )(s_orig, s_norm, t_nrm, y1, dinvr, oinvr, W2, b2.reshape(1, F_H), Wd2,
      a1.reshape(1, 1), a3.reshape(1, 1))


def _layer3_body(sa_ref, sb_ref, t2_ref, d_ref, a2_ref, y12_ref,
                 bd1_ref, bd2_ref, h2_ref, mip_ref, lmip_ref):
    d = _rowvec(d_ref)
    s = jnp.concatenate([sa_ref[...], sb_ref[...]], axis=1)
    h2 = _prelu(d * (s + t2_ref[...]), a2_ref[0, 0])
    h2_ref[...] = h2
    y12 = y12_ref[...]
    mip_ref[...] = jax.nn.sigmoid(
        jnp.sum(y12[:, :F_H] * h2, axis=-1) + bd1_ref[0, 0]).reshape(1, RB)
    lmip_ref[...] = jax.nn.sigmoid(
        jnp.sum(y12[:, F_H:] * h2, axis=-1) + bd2_ref[0, 0]).reshape(1, RB)


def _tc_layer3(s2a, s2b, t2, dinvr, a2, y12, bd1, bd2):
    full = lambda r, c: pl.BlockSpec((r, c), lambda i: (0, 0))
    blk = lambda c: pl.BlockSpec((RB, c), lambda i: (i, 0))
    row = pl.BlockSpec((1, RB), lambda i: (0, i))
    return pl.pallas_call(
        _layer3_body,
        grid=(NRB,),
        in_specs=[
            blk(F_H // 2), blk(F_H // 2), blk(F_H),
            row,
            full(1, 1), blk(2 * F_H), full(1, 1), full(1, 1),
        ],
        out_specs=[
            blk(F_H),
            row,
            row,
        ],
        out_shape=[
            jax.ShapeDtypeStruct((NPAD, F_H), jnp.float32),
            jax.ShapeDtypeStruct((1, NPAD), jnp.float32),
            jax.ShapeDtypeStruct((1, NPAD), jnp.float32),
        ],
    )(s2a, s2b, t2, dinvr, a2.reshape(1, 1), y12,
      bd1.reshape(1, 1), bd2.reshape(1, 1))


def _neg_body(g_ref, h2_ref, bd1_ref, bd2_ref, mn_ref, lmn_ref):
    gb = g_ref[0]
    h2 = h2_ref[...]
    mn_ref[...] = jax.nn.sigmoid(
        jnp.sum(gb[:, :F_H] * h2, axis=-1) + bd1_ref[0, 0]).reshape(1, 1, RB)
    lmn_ref[...] = jax.nn.sigmoid(
        jnp.sum(gb[:, F_H:] * h2, axis=-1) + bd2_ref[0, 0]).reshape(1, 1, RB)


def _tc_neg(g, h2, bd1, bd2):
    return pl.pallas_call(
        _neg_body,
        grid=(N_NEG, NRB),
        in_specs=[
            pl.BlockSpec((1, RB, 2 * F_H), lambda k, j: (k, j, 0)),
            pl.BlockSpec((RB, F_H), lambda k, j: (j, 0)),
            pl.BlockSpec((1, 1), lambda k, j: (0, 0)),
            pl.BlockSpec((1, 1), lambda k, j: (0, 0)),
        ],
        out_specs=[
            pl.BlockSpec((1, 1, RB), lambda k, j: (k, 0, j)),
            pl.BlockSpec((1, 1, RB), lambda k, j: (k, 0, j)),
        ],
        out_shape=[
            jax.ShapeDtypeStruct((N_NEG, 1, NPAD), jnp.float32),
            jax.ShapeDtypeStruct((N_NEG, 1, NPAD), jnp.float32),
        ],
    )(g, h2, bd1.reshape(1, 1), bd2.reshape(1, 1))


def _adj_body(a_ref, b_ref, o_ref):
    acc = lax.dot_general(a_ref[...], b_ref[...], (((1,), (1,)), ((), ())),
                          preferred_element_type=jnp.float32)
    o_ref[...] = jax.nn.sigmoid(acc)


def _adj_rebuilt(h2):
    return pl.pallas_call(
        _adj_body,
        grid=(pl.cdiv(N, BM), pl.cdiv(N, BN)),
        in_specs=[
            pl.BlockSpec((BM, F_H), lambda i, j: (i, 0)),
            pl.BlockSpec((BN, F_H), lambda i, j: (j, 0)),
        ],
        out_specs=pl.BlockSpec((BM, BN), lambda i, j: (i, j)),
        out_shape=jax.ShapeDtypeStruct((N, N), jnp.float32),
    )(h2, h2)


def kernel(features_norm, edge_index, neg_sample_list, W1, b1, a1, W2, b2, a2, a3, Wd1, bd1, Wd2, bd2):
    x = jnp.squeeze(features_norm, 0)
    src, dst = edge_index[0], edge_index[1]

    pad = EPAD - E
    srcp = jnp.concatenate([src, jnp.zeros((pad,), jnp.int32)]).reshape(CHT, 64)
    dstp = jnp.concatenate([dst, jnp.full((pad,), TRASH, jnp.int32)]).reshape(CHT, 64)
    srcoff = jnp.stack([srcp, srcp + NPAD])
    dst128 = dstp.reshape(EPAD // CH, CH)

    z16 = jnp.zeros((NPAD, 16), jnp.float32)
    z32 = jnp.zeros((NPAD, F_H // 2), jnp.float32)
    z64 = jnp.zeros((NPAD, F_H), jnp.float32)
    ones16 = jnp.ones((CH, 16), jnp.float32)
    xp = jnp.zeros((NPAD, F_IN), jnp.float32).at[:N].set(x)

    cnt2 = _sc_degree()(ones16, dst128, z16)
    cnt = cnt2[0, :, 0] + cnt2[1, :, 0]
    dinvr = lax.rsqrt(cnt + 1.0).reshape(1, NPAD)
    oinvr = (1.0 / jnp.maximum(cnt, 1.0)).reshape(1, NPAD)

    tab1, y1 = _tc_layer1(xp, W1, b1, Wd1, dinvr)

    s1 = _sc_spmm_split(F_H, 8)(tab1.reshape(2 * NPAD, F_H), srcoff, dstp, z64)

    tab2, t2, y12 = _tc_layer2(s1[0], s1[1], tab1[1], y1, dinvr, oinvr,
                               W2, b2, Wd2, a1, a3)

    s2 = _sc_spmm_split(F_H // 2, 8)(tab2.reshape(2 * NPAD, F_H // 2), srcoff, dstp, z32)

    h2, mip, lmip = _tc_layer3(s2[0], s2[1], t2, dinvr, a2, y12, bd1, bd2)

    negp = jnp.pad(neg_sample_list, ((0, 0), (0, NPAD - N))).reshape(NW, GCPW, GCH)
    g = _sc_gather_rows(2 * F_H)(y12, negp).reshape(N_NEG, NPAD, 2 * F_H)

    adj_rebuilt = _adj_rebuilt(h2[:N])
    mn, lmn = _tc_neg(g, h2, bd1, bd2)

    mi_pos = mip[0, :N]
    local_mi_pos = lmip[0, :N]
    mi_neg = mn[:, 0, :N]
    local_mi_neg = lmn[:, 0, :N]
    return (mi_pos, mi_neg, local_mi_pos, local_mi_neg, adj_rebuilt)


# speedup vs baseline: 9.7820x; 1.0213x over previous
import functools

import jax
import jax.numpy as jnp
from jax import lax
from jax.experimental import pallas as pl
from jax.experimental.pallas import tpu as pltpu
from jax.experimental.pallas import tpu_sc as plsc

N = 10000
E = 160000
F_IN = 128
F_H = 64
N_NEG = 5

NC, NS = 2, 16
NW = NC * NS
NPAD = 10240
TRASH = N + 100
CH = 128
EPAD = 163840
CPW = EPAD // (NW * CH)
ROWS_PT = NPAD // NS

CHT = EPAD // 64
CPS = CHT // NS
HALF = CPS // 2

RB = 2048
NRB = NPAD // RB
BM, BN = 1024, 2048


def _prelu(x, a):
    return jnp.where(x >= 0, x, a * x)


def _sc_spmm_split(D, nb):
    mesh = plsc.VectorSubcoreMesh(core_axis_name="c", subcore_axis_name="s")

    @functools.partial(
        pl.kernel,
        out_type=jax.ShapeDtypeStruct((NC, NPAD, D), jnp.float32),
        mesh=mesh,
        scratch_types=[
            pltpu.VMEM((HALF, 64), jnp.int32),
            pltpu.VMEM((HALF, 64), jnp.int32),
            pltpu.VMEM((2, nb, 64, D), jnp.float32),
            pltpu.VMEM_SHARED((NPAD, D), jnp.float32),
            pltpu.SemaphoreType.DMA,
            pltpu.SemaphoreType.DMA,
        ],
        compiler_params=pltpu.CompilerParams(use_tc_tiling_on_sc=False),
    )
    def k(table_hbm, srcoff_hbm, dst_hbm, zeros_hbm, out_hbm, srcv, dstv, rows, acc, gsem, ssem):
        cid = lax.axis_index("c")
        sid = lax.axis_index("s")
        r0 = sid * ROWS_PT
        pltpu.sync_copy(zeros_hbm.at[pl.ds(r0, ROWS_PT)], acc.at[pl.ds(r0, ROWS_PT)])
        plsc.subcore_barrier()

        c_base = sid * CPS
        ngroups = HALF // nb

        def issue_gathers(g):
            par = g % 2
            c0 = g * nb
            return [pltpu.async_copy(table_hbm.at[srcv.at[c0 + b]],
                                     rows.at[par, b], gsem)
                    for b in range(nb)]

        for h in range(2):
            pltpu.sync_copy(srcoff_hbm.at[cid, pl.ds(c_base + h * HALF, HALF)], srcv)
            pltpu.sync_copy(dst_hbm.at[pl.ds(c_base + h * HALF, HALF)], dstv)

            pend = [[], []]
            gds = issue_gathers(0)
            for g in range(ngroups):
                par = g % 2
                c0 = g * nb
                gds[0].wait()
                sds = [pltpu.async_copy(rows.at[par, 0], acc.at[dstv.at[c0]],
                                        ssem, add=True)]
                if g + 1 < ngroups:
                    for sd in pend[1 - par]:
                        sd.wait()
                    pend[1 - par] = []
                    ngds = issue_gathers(g + 1)
                for b in range(1, nb):
                    gds[b].wait()
                    sds.append(pltpu.async_copy(rows.at[par, b],
                                                acc.at[dstv.at[c0 + b]],
                                                ssem, add=True))
                pend[par] = sds
                if g + 1 < ngroups:
                    gds = ngds
            for par in (0, 1):
                for sd in pend[par]:
                    sd.wait()
        plsc.subcore_barrier()
        pltpu.sync_copy(acc.at[pl.ds(r0, ROWS_PT)], out_hbm.at[cid, pl.ds(r0, ROWS_PT)])

    return k


def _sc_degree():
    mesh = plsc.VectorSubcoreMesh(core_axis_name="c", subcore_axis_name="s")

    @functools.partial(
        pl.kernel,
        out_type=jax.ShapeDtypeStruct((NC, NPAD, 16), jnp.float32),
        mesh=mesh,
        scratch_types=[
            pltpu.VMEM((CPW, CH), jnp.int32),
            pltpu.VMEM((CH, 16), jnp.float32),
            pltpu.VMEM_SHARED((NPAD, 16), jnp.float32),
            pltpu.SemaphoreType.DMA,
        ],
        compiler_params=pltpu.CompilerParams(use_tc_tiling_on_sc=False),
    )
    def k(ones_hbm, dst_hbm, zeros_hbm, out_hbm, dstv, ones_v, acc, ssem):
        cid = lax.axis_index("c")
        sid = lax.axis_index("s")
        wid = sid * NC + cid
        r0 = sid * ROWS_PT
        pltpu.sync_copy(zeros_hbm.at[pl.ds(r0, ROWS_PT)], acc.at[pl.ds(r0, ROWS_PT)])
        pltpu.sync_copy(ones_hbm, ones_v)
        pltpu.sync_copy(dst_hbm.at[pl.ds(wid * CPW, CPW)], dstv)
        plsc.subcore_barrier()

        def group(g, _):
            sds = [pltpu.async_copy(ones_v, acc.at[dstv.at[g * 8 + b]], ssem, add=True)
                   for b in range(8)]
            for sd in sds:
                sd.wait()
            return 0

        lax.fori_loop(0, CPW // 8, group, 0)
        plsc.subcore_barrier()
        pltpu.sync_copy(acc.at[pl.ds(r0, ROWS_PT)], out_hbm.at[cid, pl.ds(r0, ROWS_PT)])

    return k


GCH = 80
GTOT = N_NEG * NPAD
GCPW = GTOT // (NW * GCH)
GNB = 5


def _sc_gather_rows(D):
    mesh = plsc.VectorSubcoreMesh(core_axis_name="c", subcore_axis_name="s")

    @functools.partial(
        pl.kernel,
        out_type=jax.ShapeDtypeStruct((GTOT, D), jnp.float32),
        mesh=mesh,
        scratch_types=[
            pltpu.VMEM((GCPW, GCH), jnp.int32),
            pltpu.VMEM((GNB, GCH, D), jnp.float32),
            pltpu.SemaphoreType.DMA,
            pltpu.SemaphoreType.DMA,
        ],
    )
    def k(table_hbm, idx_hbm, out_hbm, idxv, rows, gsem, ssem):
        cid = lax.axis_index("c")
        sid = lax.axis_index("s")
        wid = sid * NC + cid
        pltpu.sync_copy(idx_hbm.at[wid], idxv)

        def group(g, _):
            c0 = g * GNB
            gds = [pltpu.async_copy(table_hbm.at[idxv.at[c0 + b]], rows.at[b], gsem)
                   for b in range(GNB)]
            sds = []
            for b in range(GNB):
                gds[b].wait()
                sds.append(pltpu.async_copy(
                    rows.at[b],
                    out_hbm.at[pl.ds((wid * GCPW + c0 + b) * GCH, GCH)],
                    ssem))
            for sd in sds:
                sd.wait()
            return 0

        lax.fori_loop(0, GCPW // GNB, group, 0)

    return k


def _mm(a, b):
    return lax.dot_general(a, b, (((1,), (0,)), ((), ())),
                           preferred_element_type=jnp.float32)


def _rowvec(r_ref):
    return r_ref[...].reshape(RB, 1)


def _layer1_body(x_ref, w1_ref, b1_ref, wd1_ref, d_ref, tab_ref, y1_ref):
    xb = x_ref[...]
    hw1 = _mm(xb, w1_ref[...]) + b1_ref[...]
    d = _rowvec(d_ref)
    tab_ref[0] = hw1
    tab_ref[1] = hw1 * d
    y1_ref[...] = _mm(xb, wd1_ref[...])


def _tc_layer1(xp, W1, b1, Wd1, dinvr):
    return pl.pallas_call(
        _layer1_body,
        grid=(NRB,),
        in_specs=[
            pl.BlockSpec((RB, F_IN), lambda i: (i, 0)),
            pl.BlockSpec((F_IN, F_H), lambda i: (0, 0)),
            pl.BlockSpec((1, F_H), lambda i: (0, 0)),
            pl.BlockSpec((F_IN, F_H), lambda i: (0, 0)),
            pl.BlockSpec((1, RB), lambda i: (0, i)),
        ],
        out_specs=[
            pl.BlockSpec((2, RB, F_H), lambda i: (0, i, 0)),
            pl.BlockSpec((RB, F_H), lambda i: (i, 0)),
        ],
        out_shape=[
            jax.ShapeDtypeStruct((2, NPAD, F_H), jnp.float32),
            jax.ShapeDtypeStruct((NPAD, F_H), jnp.float32),
        ],
    )(xp, W1, b1.reshape(1, F_H), Wd1, dinvr)


def _layer2_body(so_ref, sn_ref, tn_ref, y1_ref, d_ref, o_ref, w2_ref, b2_ref,
                 wd2_ref, a1_ref, a3_ref, tab2_ref, t2_ref, y12_ref):
    d = _rowvec(d_ref)
    o = _rowvec(o_ref)
    h1 = _prelu(d * (sn_ref[...] + tn_ref[...]), a1_ref[0, 0])
    hn = _prelu(o * so_ref[...], a3_ref[0, 0])
    hw2 = _mm(h1, w2_ref[...]) + b2_ref[...]
    t2 = hw2 * d
    tab2_ref[0] = t2[:, :F_H // 2]
    tab2_ref[1] = t2[:, F_H // 2:]
    t2_ref[...] = t2
    y12_ref[...] = jnp.concatenate(
        [y1_ref[...], _mm(hn, wd2_ref[...])], axis=1)


def _tc_layer2(s_orig, s_norm, t_nrm, y1, dinvr, oinvr, W2, b2, Wd2, a1, a3):
    full = lambda r, c: pl.BlockSpec((r, c), lambda i: (0, 0))
    blk = lambda c: pl.BlockSpec((RB, c), lambda i: (i, 0))
    row = pl.BlockSpec((1, RB), lambda i: (0, i))
    return pl.pallas_call(
        _layer2_body,
        grid=(NRB,),
        in_specs=[
            blk(F_H), blk(F_H), blk(F_H), blk(F_H),
            row, row,
            full(F_H, F_H), full(1, F_H), full(F_H, F_H),
            full(1, 1), full(1, 1),
        ],
        out_specs=[
            pl.BlockSpec((2, RB, F_H // 2), lambda i: (0, i, 0)),
            blk(F_H), blk(2 * F_H),
        ],
        out_shape=[
            jax.ShapeDtypeStruct((2, NPAD, F_H // 2), jnp.float32),
            jax.ShapeDtypeStruct((NPAD, F_H), jnp.float32),
            jax.ShapeDtypeStruct((NPAD, 2 * F_H), jnp.float32),
        ],
    )(s_orig, s_norm, t_nrm, y1, dinvr, oinvr, W2, b2.reshape(1, F_H), Wd2,
      a1.reshape(1, 1), a3.reshape(1, 1))


def _layer3_body(sa_ref, sb_ref, t2_ref, d_ref, a2_ref, y12_ref,
                 bd1_ref, bd2_ref, h2_ref, mip_ref, lmip_ref):
    d = _rowvec(d_ref)
    s = jnp.concatenate([sa_ref[...], sb_ref[...]], axis=1)
    h2 = _prelu(d * (s + t2_ref[...]), a2_ref[0, 0])
    h2_ref[...] = h2
    y12 = y12_ref[...]
    mip_ref[...] = jax.nn.sigmoid(
        jnp.sum(y12[:, :F_H] * h2, axis=-1) + bd1_ref[0, 0]).reshape(1, RB)
    lmip_ref[...] = jax.nn.sigmoid(
        jnp.sum(y12[:, F_H:] * h2, axis=-1) + bd2_ref[0, 0]).reshape(1, RB)


def _tc_layer3(s2a, s2b, t2, dinvr, a2, y12, bd1, bd2):
    full = lambda r, c: pl.BlockSpec((r, c), lambda i: (0, 0))
    blk = lambda c: pl.BlockSpec((RB, c), lambda i: (i, 0))
    row = pl.BlockSpec((1, RB), lambda i: (0, i))
    return pl.pallas_call(
        _layer3_body,
        grid=(NRB,),
        in_specs=[
            blk(F_H // 2), blk(F_H // 2), blk(F_H),
            row,
            full(1, 1), blk(2 * F_H), full(1, 1), full(1, 1),
        ],
        out_specs=[
            blk(F_H),
            row,
            row,
        ],
        out_shape=[
            jax.ShapeDtypeStruct((NPAD, F_H), jnp.float32),
            jax.ShapeDtypeStruct((1, NPAD), jnp.float32),
            jax.ShapeDtypeStruct((1, NPAD), jnp.float32),
        ],
    )(s2a, s2b, t2, dinvr, a2.reshape(1, 1), y12,
      bd1.reshape(1, 1), bd2.reshape(1, 1))


def _neg_body(g_ref, h2_ref, bd1_ref, bd2_ref, mn_ref, lmn_ref):
    gb = g_ref[0]
    h2 = h2_ref[...]
    mn_ref[...] = jax.nn.sigmoid(
        jnp.sum(gb[:, :F_H] * h2, axis=-1) + bd1_ref[0, 0]).reshape(1, 1, RB)
    lmn_ref[...] = jax.nn.sigmoid(
        jnp.sum(gb[:, F_H:] * h2, axis=-1) + bd2_ref[0, 0]).reshape(1, 1, RB)


def _tc_neg(g, h2, bd1, bd2):
    return pl.pallas_call(
        _neg_body,
        grid=(N_NEG, NRB),
        in_specs=[
            pl.BlockSpec((1, RB, 2 * F_H), lambda k, j: (k, j, 0)),
            pl.BlockSpec((RB, F_H), lambda k, j: (j, 0)),
            pl.BlockSpec((1, 1), lambda k, j: (0, 0)),
            pl.BlockSpec((1, 1), lambda k, j: (0, 0)),
        ],
        out_specs=[
            pl.BlockSpec((1, 1, RB), lambda k, j: (k, 0, j)),
            pl.BlockSpec((1, 1, RB), lambda k, j: (k, 0, j)),
        ],
        out_shape=[
            jax.ShapeDtypeStruct((N_NEG, 1, NPAD), jnp.float32),
            jax.ShapeDtypeStruct((N_NEG, 1, NPAD), jnp.float32),
        ],
    )(g, h2, bd1.reshape(1, 1), bd2.reshape(1, 1))


def _adj_body(a_ref, b_ref, o_ref):
    acc = lax.dot_general(a_ref[...], b_ref[...], (((1,), (1,)), ((), ())),
                          preferred_element_type=jnp.float32)
    o_ref[...] = jax.nn.sigmoid(acc)


def _adj_rebuilt(h2):
    return pl.pallas_call(
        _adj_body,
        grid=(pl.cdiv(N, BM), pl.cdiv(N, BN)),
        in_specs=[
            pl.BlockSpec((BM, F_H), lambda i, j: (i, 0)),
            pl.BlockSpec((BN, F_H), lambda i, j: (j, 0)),
        ],
        out_specs=pl.BlockSpec((BM, BN), lambda i, j: (i, j)),
        out_shape=jax.ShapeDtypeStruct((N, N), jnp.float32),
    )(h2, h2)


def kernel(features_norm, edge_index, neg_sample_list, W1, b1, a1, W2, b2, a2, a3, Wd1, bd1, Wd2, bd2):
    x = jnp.squeeze(features_norm, 0)
    src, dst = edge_index[0], edge_index[1]

    pad = EPAD - E
    srcp = jnp.concatenate([src, jnp.zeros((pad,), jnp.int32)]).reshape(CHT, 64)
    dstp = jnp.concatenate([dst, jnp.full((pad,), TRASH, jnp.int32)]).reshape(CHT, 64)
    srcoff = jnp.stack([srcp, srcp + NPAD])
    dst128 = dstp.reshape(EPAD // CH, CH)

    z16 = jnp.zeros((NPAD, 16), jnp.float32)
    z32 = jnp.zeros((NPAD, F_H // 2), jnp.float32)
    z64 = jnp.zeros((NPAD, F_H), jnp.float32)
    ones16 = jnp.ones((CH, 16), jnp.float32)
    xp = jnp.zeros((NPAD, F_IN), jnp.float32).at[:N].set(x)

    cnt2 = _sc_degree()(ones16, dst128, z16)
    cnt = cnt2[0, :, 0] + cnt2[1, :, 0]
    dinvr = lax.rsqrt(cnt + 1.0).reshape(1, NPAD)
    oinvr = (1.0 / jnp.maximum(cnt, 1.0)).reshape(1, NPAD)

    tab1, y1 = _tc_layer1(xp, W1, b1, Wd1, dinvr)

    s1 = _sc_spmm_split(F_H, 8)(tab1.reshape(2 * NPAD, F_H), srcoff, dstp, z64)

    tab2, t2, y12 = _tc_layer2(s1[0], s1[1], tab1[1], y1, dinvr, oinvr,
                               W2, b2, Wd2, a1, a3)

    s2 = _sc_spmm_split(F_H // 2, 8)(tab2.reshape(2 * NPAD, F_H // 2), srcoff, dstp, z32)

    h2, mip, lmip = _tc_layer3(s2[0], s2[1], t2, dinvr, a2, y12, bd1, bd2)

    negp = jnp.pad(neg_sample_list, ((0, 0), (0, NPAD - N))).reshape(NW, GCPW, GCH)
    g = _sc_gather_rows(2 * F_H)(y12, negp).reshape(N_NEG, NPAD, 2 * F_H)

    adj_rebuilt = _adj_rebuilt(h2[:N])
    mn, lmn = _tc_neg(g, h2, bd1, bd2)

    mi_pos = mip[0, :N]
    local_mi_pos = lmip[0, :N]
    mi_neg = mn[:, 0, :N]
    local_mi_neg = lmn[:, 0, :N]
    return (mi_pos, mi_neg, local_mi_pos, local_mi_neg, adj_rebuilt)


# speedup vs baseline: 9.7882x; 1.0006x over previous
import functools

import jax
import jax.numpy as jnp
from jax import lax
from jax.experimental import pallas as pl
from jax.experimental.pallas import tpu as pltpu
from jax.experimental.pallas import tpu_sc as plsc

N = 10000
E = 160000
F_IN = 128
F_H = 64
N_NEG = 5

NC, NS = 2, 16
NW = NC * NS
NPAD = 10240
TRASH = N + 100
CH = 128
EPAD = 163840
CPW = EPAD // (NW * CH)
ROWS_PT = NPAD // NS

CHT = EPAD // 64
CPS = CHT // NS
HALF = CPS // 2

RB = 2048
NRB = NPAD // RB
BM, BN = 1024, 2048


def _prelu(x, a):
    return jnp.where(x >= 0, x, a * x)


def _sc_spmm_split(D, nb):
    mesh = plsc.VectorSubcoreMesh(core_axis_name="c", subcore_axis_name="s")

    @functools.partial(
        pl.kernel,
        out_type=jax.ShapeDtypeStruct((NC, NPAD, D), jnp.float32),
        mesh=mesh,
        scratch_types=[
            pltpu.VMEM((HALF, 64), jnp.int32),
            pltpu.VMEM((HALF, 64), jnp.int32),
            pltpu.VMEM((2, nb, 64, D), jnp.float32),
            pltpu.VMEM_SHARED((NPAD, D), jnp.float32),
            pltpu.SemaphoreType.DMA,
            pltpu.SemaphoreType.DMA,
        ],
        compiler_params=pltpu.CompilerParams(use_tc_tiling_on_sc=False),
    )
    def k(table_hbm, srcoff_hbm, dst_hbm, zeros_hbm, out_hbm, srcv, dstv, rows, acc, gsem, ssem):
        cid = lax.axis_index("c")
        sid = lax.axis_index("s")
        r0 = sid * ROWS_PT
        pltpu.sync_copy(zeros_hbm.at[pl.ds(r0, ROWS_PT)], acc.at[pl.ds(r0, ROWS_PT)])
        plsc.subcore_barrier()

        c_base = sid * CPS
        ngroups = HALF // nb

        def issue_gathers(g):
            par = g % 2
            c0 = g * nb
            return [pltpu.async_copy(table_hbm.at[srcv.at[c0 + b]],
                                     rows.at[par, b], gsem)
                    for b in range(nb)]

        for h in range(2):
            pltpu.sync_copy(srcoff_hbm.at[cid, pl.ds(c_base + h * HALF, HALF)], srcv)
            pltpu.sync_copy(dst_hbm.at[pl.ds(c_base + h * HALF, HALF)], dstv)

            pend = [[], []]
            gds = issue_gathers(0)
            for g in range(ngroups):
                par = g % 2
                c0 = g * nb
                gds[0].wait()
                sds = [pltpu.async_copy(rows.at[par, 0], acc.at[dstv.at[c0]],
                                        ssem, add=True)]
                if g + 1 < ngroups:
                    for sd in pend[1 - par]:
                        sd.wait()
                    pend[1 - par] = []
                    ngds = issue_gathers(g + 1)
                for b in range(1, nb):
                    gds[b].wait()
                    sds.append(pltpu.async_copy(rows.at[par, b],
                                                acc.at[dstv.at[c0 + b]],
                                                ssem, add=True))
                pend[par] = sds
                if g + 1 < ngroups:
                    gds = ngds
            for par in (0, 1):
                for sd in pend[par]:
                    sd.wait()
        plsc.subcore_barrier()
        pltpu.sync_copy(acc.at[pl.ds(r0, ROWS_PT)], out_hbm.at[cid, pl.ds(r0, ROWS_PT)])

    return k


def _sc_degree():
    mesh = plsc.VectorSubcoreMesh(core_axis_name="c", subcore_axis_name="s")

    @functools.partial(
        pl.kernel,
        out_type=jax.ShapeDtypeStruct((NC, NPAD, 16), jnp.float32),
        mesh=mesh,
        scratch_types=[
            pltpu.VMEM((CPW, CH), jnp.int32),
            pltpu.VMEM((CH, 16), jnp.float32),
            pltpu.VMEM_SHARED((NPAD, 16), jnp.float32),
            pltpu.SemaphoreType.DMA,
        ],
        compiler_params=pltpu.CompilerParams(use_tc_tiling_on_sc=False),
    )
    def k(ones_hbm, dst_hbm, zeros_hbm, out_hbm, dstv, ones_v, acc, ssem):
        cid = lax.axis_index("c")
        sid = lax.axis_index("s")
        wid = sid * NC + cid
        r0 = sid * ROWS_PT
        pltpu.sync_copy(zeros_hbm.at[pl.ds(r0, ROWS_PT)], acc.at[pl.ds(r0, ROWS_PT)])
        pltpu.sync_copy(ones_hbm, ones_v)
        pltpu.sync_copy(dst_hbm.at[pl.ds(wid * CPW, CPW)], dstv)
        plsc.subcore_barrier()

        def group(g, _):
            sds = [pltpu.async_copy(ones_v, acc.at[dstv.at[g * 8 + b]], ssem, add=True)
                   for b in range(8)]
            for sd in sds:
                sd.wait()
            return 0

        lax.fori_loop(0, CPW // 8, group, 0)
        plsc.subcore_barrier()
        pltpu.sync_copy(acc.at[pl.ds(r0, ROWS_PT)], out_hbm.at[cid, pl.ds(r0, ROWS_PT)])

    return k


GCH = 80
GTOT = N_NEG * NPAD
GCPW = GTOT // (NW * GCH)
GNB = 5


def _sc_gather_rows(D):
    mesh = plsc.VectorSubcoreMesh(core_axis_name="c", subcore_axis_name="s")

    @functools.partial(
        pl.kernel,
        out_type=jax.ShapeDtypeStruct((GTOT, D), jnp.float32),
        mesh=mesh,
        scratch_types=[
            pltpu.VMEM((GCPW, GCH), jnp.int32),
            pltpu.VMEM((GNB, GCH, D), jnp.float32),
            pltpu.SemaphoreType.DMA,
            pltpu.SemaphoreType.DMA,
        ],
    )
    def k(table_hbm, idx_hbm, out_hbm, idxv, rows, gsem, ssem):
        cid = lax.axis_index("c")
        sid = lax.axis_index("s")
        wid = sid * NC + cid
        pltpu.sync_copy(idx_hbm.at[wid], idxv)

        def group(g, _):
            c0 = g * GNB
            gds = [pltpu.async_copy(table_hbm.at[idxv.at[c0 + b]], rows.at[b], gsem)
                   for b in range(GNB)]
            sds = []
            for b in range(GNB):
                gds[b].wait()
                sds.append(pltpu.async_copy(
                    rows.at[b],
                    out_hbm.at[pl.ds((wid * GCPW + c0 + b) * GCH, GCH)],
                    ssem))
            for sd in sds:
                sd.wait()
            return 0

        lax.fori_loop(0, GCPW // GNB, group, 0)

    return k


def _mm(a, b):
    return lax.dot_general(a, b, (((1,), (0,)), ((), ())),
                           preferred_element_type=jnp.float32)


def _rowvec(r_ref):
    return r_ref[...].reshape(RB, 1)


def _layer1_body(x_ref, w1_ref, b1_ref, wd1_ref, d_ref, tab_ref, y1_ref):
    xb = x_ref[...]
    hw1 = _mm(xb, w1_ref[...]) + b1_ref[...]
    d = _rowvec(d_ref)
    tab_ref[0] = hw1
    tab_ref[1] = hw1 * d
    y1_ref[...] = _mm(xb, wd1_ref[...])


def _tc_layer1(xp, W1, b1, Wd1, dinvr):
    return pl.pallas_call(
        _layer1_body,
        grid=(NRB,),
        in_specs=[
            pl.BlockSpec((RB, F_IN), lambda i: (i, 0)),
            pl.BlockSpec((F_IN, F_H), lambda i: (0, 0)),
            pl.BlockSpec((1, F_H), lambda i: (0, 0)),
            pl.BlockSpec((F_IN, F_H), lambda i: (0, 0)),
            pl.BlockSpec((1, RB), lambda i: (0, i)),
        ],
        out_specs=[
            pl.BlockSpec((2, RB, F_H), lambda i: (0, i, 0)),
            pl.BlockSpec((RB, F_H), lambda i: (i, 0)),
        ],
        out_shape=[
            jax.ShapeDtypeStruct((2, NPAD, F_H), jnp.float32),
            jax.ShapeDtypeStruct((NPAD, F_H), jnp.float32),
        ],
    )(xp, W1, b1.reshape(1, F_H), Wd1, dinvr)


def _layer2_body(so_ref, sn_ref, tn_ref, y1_ref, d_ref, o_ref, w2_ref, b2_ref,
                 wd2_ref, a1_ref, a3_ref, tab2_ref, t2_ref, y12_ref):
    d = _rowvec(d_ref)
    o = _rowvec(o_ref)
    h1 = _prelu(d * (sn_ref[...] + tn_ref[...]), a1_ref[0, 0])
    hn = _prelu(o * so_ref[...], a3_ref[0, 0])
    hw2 = _mm(h1, w2_ref[...]) + b2_ref[...]
    t2 = hw2 * d
    tab2_ref[0] = t2[:, :F_H // 2]
    tab2_ref[1] = t2[:, F_H // 2:]
    t2_ref[...] = t2
    y12_ref[...] = jnp.concatenate(
        [y1_ref[...], _mm(hn, wd2_ref[...])], axis=1)


def _tc_layer2(s_orig, s_norm, t_nrm, y1, dinvr, oinvr, W2, b2, Wd2, a1, a3):
    full = lambda r, c: pl.BlockSpec((r, c), lambda i: (0, 0))
    blk = lambda c: pl.BlockSpec((RB, c), lambda i: (i, 0))
    row = pl.BlockSpec((1, RB), lambda i: (0, i))
    return pl.pallas_call(
        _layer2_body,
        grid=(NRB,),
        in_specs=[
            blk(F_H), blk(F_H), blk(F_H), blk(F_H),
            row, row,
            full(F_H, F_H), full(1, F_H), full(F_H, F_H),
            full(1, 1), full(1, 1),
        ],
        out_specs=[
            pl.BlockSpec((2, RB, F_H // 2), lambda i: (0, i, 0)),
            blk(F_H), blk(2 * F_H),
        ],
        out_shape=[
            jax.ShapeDtypeStruct((2, NPAD, F_H // 2), jnp.float32),
            jax.ShapeDtypeStruct((NPAD, F_H), jnp.float32),
            jax.ShapeDtypeStruct((NPAD, 2 * F_H), jnp.float32),
        ],
    )(s_orig, s_norm, t_nrm, y1, dinvr, oinvr, W2, b2.reshape(1, F_H), Wd2,
      a1.reshape(1, 1), a3.reshape(1, 1))


def _layer3_body(sa_ref, sb_ref, t2_ref, d_ref, a2_ref, y12_ref,
                 bd1_ref, bd2_ref, h2_ref, mip_ref, lmip_ref):
    d = _rowvec(d_ref)
    s = jnp.concatenate([sa_ref[...], sb_ref[...]], axis=1)
    h2 = _prelu(d * (s + t2_ref[...]), a2_ref[0, 0])
    h2_ref[...] = h2
    y12 = y12_ref[...]
    mip_ref[...] = jax.nn.sigmoid(
        jnp.sum(y12[:, :F_H] * h2, axis=-1) + bd1_ref[0, 0]).reshape(1, RB)
    lmip_ref[...] = jax.nn.sigmoid(
        jnp.sum(y12[:, F_H:] * h2, axis=-1) + bd2_ref[0, 0]).reshape(1, RB)


def _tc_layer3(s2a, s2b, t2, dinvr, a2, y12, bd1, bd2):
    full = lambda r, c: pl.BlockSpec((r, c), lambda i: (0, 0))
    blk = lambda c: pl.BlockSpec((RB, c), lambda i: (i, 0))
    row = pl.BlockSpec((1, RB), lambda i: (0, i))
    return pl.pallas_call(
        _layer3_body,
        grid=(NRB,),
        in_specs=[
            blk(F_H // 2), blk(F_H // 2), blk(F_H),
            row,
            full(1, 1), blk(2 * F_H), full(1, 1), full(1, 1),
        ],
        out_specs=[
            blk(F_H),
            row,
            row,
        ],
        out_shape=[
            jax.ShapeDtypeStruct((NPAD, F_H), jnp.float32),
            jax.ShapeDtypeStruct((1, NPAD), jnp.float32),
            jax.ShapeDtypeStruct((1, NPAD), jnp.float32),
        ],
    )(s2a, s2b, t2, dinvr, a2.reshape(1, 1), y12,
      bd1.reshape(1, 1), bd2.reshape(1, 1))


def _neg_body(g_ref, h2_ref, bd1_ref, bd2_ref, mn_ref, lmn_ref):
    gb = g_ref[0]
    h2 = h2_ref[...]
    mn_ref[...] = jax.nn.sigmoid(
        jnp.sum(gb[:, :F_H] * h2, axis=-1) + bd1_ref[0, 0]).reshape(1, 1, RB)
    lmn_ref[...] = jax.nn.sigmoid(
        jnp.sum(gb[:, F_H:] * h2, axis=-1) + bd2_ref[0, 0]).reshape(1, 1, RB)


def _tc_neg(g, h2, bd1, bd2):
    return pl.pallas_call(
        _neg_body,
        grid=(N_NEG, NRB),
        in_specs=[
            pl.BlockSpec((1, RB, 2 * F_H), lambda k, j: (k, j, 0)),
            pl.BlockSpec((RB, F_H), lambda k, j: (j, 0)),
            pl.BlockSpec((1, 1), lambda k, j: (0, 0)),
            pl.BlockSpec((1, 1), lambda k, j: (0, 0)),
        ],
        out_specs=[
            pl.BlockSpec((1, 1, RB), lambda k, j: (k, 0, j)),
            pl.BlockSpec((1, 1, RB), lambda k, j: (k, 0, j)),
        ],
        out_shape=[
            jax.ShapeDtypeStruct((N_NEG, 1, NPAD), jnp.float32),
            jax.ShapeDtypeStruct((N_NEG, 1, NPAD), jnp.float32),
        ],
    )(g, h2, bd1.reshape(1, 1), bd2.reshape(1, 1))


def _adj_body(a_ref, b_ref, o_ref):
    acc = lax.dot_general(a_ref[...], b_ref[...], (((1,), (1,)), ((), ())),
                          preferred_element_type=jnp.float32)
    o_ref[...] = jax.nn.sigmoid(acc)


def _adj_rebuilt(h2):
    return pl.pallas_call(
        _adj_body,
        grid=(pl.cdiv(N, BM), pl.cdiv(N, BN)),
        in_specs=[
            pl.BlockSpec((BM, F_H), lambda i, j: (i, 0)),
            pl.BlockSpec((BN, F_H), lambda i, j: (j, 0)),
        ],
        out_specs=pl.BlockSpec((BM, BN), lambda i, j: (i, j)),
        out_shape=jax.ShapeDtypeStruct((N, N), jnp.float32),
    )(h2, h2)


def kernel(features_norm, edge_index, neg_sample_list, W1, b1, a1, W2, b2, a2, a3, Wd1, bd1, Wd2, bd2):
    x = jnp.squeeze(features_norm, 0)
    src, dst = edge_index[0], edge_index[1]

    pad = EPAD - E
    srcp = jnp.concatenate([src, jnp.zeros((pad,), jnp.int32)]).reshape(CHT, 64)
    dstp = jnp.concatenate([dst, jnp.full((pad,), TRASH, jnp.int32)]).reshape(CHT, 64)
    srcoff = jnp.stack([srcp, srcp + NPAD])
    dst128 = dstp.reshape(EPAD // CH, CH)

    z16 = jnp.zeros((NPAD, 16), jnp.float32)
    z32 = jnp.zeros((NPAD, F_H // 2), jnp.float32)
    z64 = jnp.zeros((NPAD, F_H), jnp.float32)
    ones16 = jnp.ones((CH, 16), jnp.float32)
    xp = jnp.zeros((NPAD, F_IN), jnp.float32).at[:N].set(x)

    cnt2 = _sc_degree()(ones16, dst128, z16)
    cnt = cnt2[0, :, 0] + cnt2[1, :, 0]
    dinvr = lax.rsqrt(cnt + 1.0).reshape(1, NPAD)
    oinvr = (1.0 / jnp.maximum(cnt, 1.0)).reshape(1, NPAD)

    tab1, y1 = _tc_layer1(xp, W1, b1, Wd1, dinvr)

    s1 = _sc_spmm_split(F_H, 8)(tab1.reshape(2 * NPAD, F_H), srcoff, dstp, z64)

    tab2, t2, y12 = _tc_layer2(s1[0], s1[1], tab1[1], y1, dinvr, oinvr,
                               W2, b2, Wd2, a1, a3)

    s2 = _sc_spmm_split(F_H // 2, 16)(tab2.reshape(2 * NPAD, F_H // 2), srcoff, dstp, z32)

    h2, mip, lmip = _tc_layer3(s2[0], s2[1], t2, dinvr, a2, y12, bd1, bd2)

    negp = jnp.pad(neg_sample_list, ((0, 0), (0, NPAD - N))).reshape(NW, GCPW, GCH)
    g = _sc_gather_rows(2 * F_H)(y12, negp).reshape(N_NEG, NPAD, 2 * F_H)

    adj_rebuilt = _adj_rebuilt(h2[:N])
    mn, lmn = _tc_neg(g, h2, bd1, bd2)

    mi_pos = mip[0, :N]
    local_mi_pos = lmip[0, :N]
    mi_neg = mn[:, 0, :N]
    local_mi_neg = lmn[:, 0, :N]
    return (mi_pos, mi_neg, local_mi_pos, local_mi_neg, adj_rebuilt)
